# Initial kernel scaffold; baseline (speedup 1.0000x reference)
import jax
import jax.numpy as jnp
from jax.experimental import pallas as pl


def kernel(x, edge_w, edge_index, batch, params):
    raise NotImplementedError("write your pallas kernel here")


# speedup vs baseline: 3.2927x; 3.2927x over previous
import functools

import jax
import jax.numpy as jnp
from jax import lax
from jax.experimental import pallas as pl
from jax.experimental.pallas import tpu as pltpu
from jax.experimental.pallas import tpu_sc as plsc

N_NODES = 10000
N_EDGES = 320000
D = 128
H = D // 2
NS = 16
L = 16

EPC = N_EDGES // NS
BE = 80
NB = EPC // BE
EPC2 = N_EDGES // (2 * NS)
NB2 = EPC2 // BE
NP = 10240
RPC = NP // NS
RC = 32
NRC = RPC // RC

_EPS = 1e-16


_R = 2000
_RE = 2000


def _h_from_parts(uh, sasb_lo, sasb_hi, deg0, deg1):
    sa = jnp.concatenate([sasb_lo[:, :H], sasb_hi[:, :H]], axis=1)
    sb = jnp.concatenate([sasb_lo[:, H:], sasb_hi[:, H:]], axis=1)
    h = jnp.maximum(uh + sa / (sb + _EPS), 0.0)
    deg = deg0[:, :1] + deg1[:, :1]
    return jnp.where(deg > 0.0, h, 0.0)


def _mm(h, w_ref, b_ref):
    return jnp.dot(h, w_ref[...], preferred_element_type=jnp.float32) + b_ref[...]


def _emit_node_outs(h, vw, vb, aw, ab, bw, bb, ah_o, bvlo_o, bvhi_o):
    vh = _mm(h, vw, vb)
    ah = _mm(h, aw, ab)
    bh = _mm(h, bw, bb)
    ah_o[...] = ah
    bvlo_o[...] = jnp.concatenate([bh[:, :H], vh[:, :H]], axis=1)
    bvhi_o[...] = jnp.concatenate([bh[:, H:], vh[:, H:]], axis=1)


def _node_first_body(x_ref, ew_ref, eb_ref, uw_ref, ub_ref, vw_ref, vb_ref,
                     aw_ref, ab_ref, bw_ref, bb_ref,
                     uh_o, ah_o, bvlo_o, bvhi_o):
    h = _mm(x_ref[...], ew_ref, eb_ref)
    uh_o[...] = _mm(h, uw_ref, ub_ref)
    _emit_node_outs(h, vw_ref, vb_ref, aw_ref, ab_ref, bw_ref, bb_ref,
                    ah_o, bvlo_o, bvhi_o)


def _node_rest_body(uhp_ref, sasblo_ref, sasbhi_ref, deg0_ref, deg1_ref,
                    uw_ref, ub_ref, vw_ref, vb_ref, aw_ref, ab_ref,
                    bw_ref, bb_ref,
                    uh_o, ah_o, bvlo_o, bvhi_o):
    h = _h_from_parts(uhp_ref[...], sasblo_ref[...], sasbhi_ref[...],
                      deg0_ref[...], deg1_ref[...])
    uh_o[...] = _mm(h, uw_ref, ub_ref)
    _emit_node_outs(h, vw_ref, vb_ref, aw_ref, ab_ref, bw_ref, bb_ref,
                    ah_o, bvlo_o, bvhi_o)


def _edge_first_body(ew_ref, wt_ref, eb_ref, ct_ref, cb_ref, celo_o, cehi_o):
    e0 = ew_ref[...] * wt_ref[...] + eb_ref[...]
    ce = _mm(e0, ct_ref, cb_ref)
    celo_o[...] = ce[:, :H]
    cehi_o[...] = ce[:, H:]


def _edge_rest_body(elo_ref, ehi_ref, ct_ref, cb_ref, celo_o, cehi_o):
    e = jnp.concatenate([elo_ref[...], ehi_ref[...]], axis=1)
    ce = _mm(e, ct_ref, cb_ref)
    celo_o[...] = ce[:, :H]
    cehi_o[...] = ce[:, H:]


def _fin_body(uhp_ref, sasblo_ref, sasbhi_ref, deg0_ref, deg1_ref, out_o):
    i = pl.program_id(0)
    h = _h_from_parts(uhp_ref[...], sasblo_ref[...], sasbhi_ref[...],
                      deg0_ref[...], deg1_ref[...])
    part = jnp.sum(h, axis=0, keepdims=True)

    @pl.when(i == 0)
    def _():
        out_o[...] = part

    @pl.when(i > 0)
    def _():
        out_o[...] = out_o[...] + part

    @pl.when(i == (N_NODES // _R) - 1)
    def _():
        out_o[...] = out_o[...] * (1.0 / N_NODES)


def _full_spec(shape):
    return pl.BlockSpec(shape, lambda i: (0, 0))


def _row_spec(block_rows, cols):
    return pl.BlockSpec((block_rows, cols), lambda i: (i, 0))


_W = _full_spec((D, D))
_B = _full_spec((1, D))

_NODE_OUTS = (
    jax.ShapeDtypeStruct((N_NODES, D), jnp.float32),
    jax.ShapeDtypeStruct((N_NODES, D), jnp.float32),
    jax.ShapeDtypeStruct((N_NODES, D), jnp.float32),
    jax.ShapeDtypeStruct((N_NODES, D), jnp.float32),
)
_NODE_OUT_SPECS = (_row_spec(_R, D),) * 4

_node_first = pl.pallas_call(
    _node_first_body,
    grid=(N_NODES // _R,),
    in_specs=[_row_spec(_R, D), _W, _B, _W, _B, _W, _B, _W, _B, _W, _B],
    out_specs=_NODE_OUT_SPECS,
    out_shape=_NODE_OUTS,
)

_node_rest = pl.pallas_call(
    _node_rest_body,
    grid=(N_NODES // _R,),
    in_specs=[_row_spec(_R, D),
              _row_spec(_R, D), _row_spec(_R, D),
              _row_spec(_R, D), _row_spec(_R, D),
              _W, _B, _W, _B, _W, _B, _W, _B],
    out_specs=_NODE_OUT_SPECS,
    out_shape=_NODE_OUTS,
)

_EDGE_OUTS = (
    jax.ShapeDtypeStruct((N_EDGES, H), jnp.float32),
    jax.ShapeDtypeStruct((N_EDGES, H), jnp.float32),
)

_edge_first = pl.pallas_call(
    _edge_first_body,
    grid=(N_EDGES // _RE,),
    in_specs=[_row_spec(_RE, 1), _full_spec((1, D)), _B, _W, _B],
    out_specs=(_row_spec(_RE, H), _row_spec(_RE, H)),
    out_shape=_EDGE_OUTS,
)

_edge_rest = pl.pallas_call(
    _edge_rest_body,
    grid=(N_EDGES // _RE,),
    in_specs=[_row_spec(_RE, H), _row_spec(_RE, H), _W, _B],
    out_specs=(_row_spec(_RE, H), _row_spec(_RE, H)),
    out_shape=_EDGE_OUTS,
)

_fin = pl.pallas_call(
    _fin_body,
    grid=(N_NODES // _R,),
    in_specs=[_row_spec(_R, D),
              _row_spec(_R, D), _row_spec(_R, D),
              _row_spec(_R, D), _row_spec(_R, D)],
    out_specs=pl.BlockSpec((1, D), lambda i: (0, 0)),
    out_shape=jax.ShapeDtypeStruct((1, D), jnp.float32),
)


_f32 = jnp.float32


def _zero_stage(stage, cols):
    def zrow(r, _):
        for k in range(cols // L):
            stage[r, pl.ds(k * L, L)] = jnp.zeros((L,), _f32)
        return _
    lax.fori_loop(0, RC, zrow, 0)


@functools.cache
def _make_sc_edge_kernel():
    mesh = plsc.VectorSubcoreMesh(core_axis_name="c", subcore_axis_name="s")
    outs = (
        jax.ShapeDtypeStruct((NP, D), _f32),
        jax.ShapeDtypeStruct((NP, D), _f32),
        jax.ShapeDtypeStruct((N_EDGES, H), _f32),
        jax.ShapeDtypeStruct((N_EDGES, H), _f32),
    )
    scratch = [
        pltpu.VMEM((BE,), jnp.int32),
        pltpu.VMEM((BE,), jnp.int32),
        pltpu.VMEM((BE, D), _f32),
        pltpu.VMEM((BE, D), _f32),
        pltpu.VMEM((BE, H), _f32),
        pltpu.VMEM((RC, D), _f32),
        pltpu.VMEM_SHARED((NP, D), _f32),
        pltpu.SemaphoreType.DMA,
        pltpu.SemaphoreType.DMA,
    ]

    @functools.partial(pl.kernel, out_type=outs, mesh=mesh,
                       scratch_types=scratch)
    def sck(celo, cehi, ah, bvlo, bvhi, src, dst,
            sasblo_o, sasbhi_o, elo_o, ehi_o,
            idxs, idxd, ar, bvr, cer, stage, sasb_sh,
            sem1, sem2):
        c = lax.axis_index("c")
        s = lax.axis_index("s")
        row0 = s * RPC
        base0 = s * EPC

        def run_half(ce_h, bv_h, sasb_o, e_o, col0):
            _zero_stage(stage, D)
            for j in range(NRC):
                pltpu.sync_copy(stage, sasb_sh.at[pl.ds(row0 + j * RC, RC)])
            plsc.subcore_barrier()

            def blk(t, _):
                base = base0 + t * BE
                pltpu.sync_copy(src.at[pl.ds(base, BE)], idxs)
                pltpu.sync_copy(dst.at[pl.ds(base, BE)], idxd)
                cp1 = pltpu.async_copy(ah.at[idxd], ar, sem1)
                cp2 = pltpu.async_copy(bv_h.at[idxs], bvr, sem2)
                pltpu.sync_copy(ce_h.at[pl.ds(base, BE)], cer)
                cp1.wait()
                cp2.wait()

                def row(r, _):
                    for k in range(H // L):
                        a_k = ar[r, pl.ds(col0 + k * L, L)]
                        b_k = bvr[r, pl.ds(k * L, L)]
                        v_k = bvr[r, pl.ds(H + k * L, L)]
                        ce_k = cer[r, pl.ds(k * L, L)]
                        eij = a_k + b_k + ce_k
                        sg = 1.0 / (1.0 + jnp.exp(-eij))
                        cer[r, pl.ds(k * L, L)] = jnp.maximum(eij, 0.0)
                        ar[r, pl.ds(k * L, L)] = v_k * sg
                        ar[r, pl.ds(H + k * L, L)] = sg
                    return _
                lax.fori_loop(0, BE, row, 0)

                pltpu.sync_copy(ar, sasb_sh.at[idxd], add=True)
                pltpu.sync_copy(cer, e_o.at[pl.ds(base, BE)])
                return _
            lax.fori_loop(0, NB, blk, 0)
            plsc.subcore_barrier()

            for j in range(NRC):
                r0 = row0 + j * RC
                pltpu.sync_copy(sasb_sh.at[pl.ds(r0, RC)], stage)
                pltpu.sync_copy(stage, sasb_o.at[pl.ds(r0, RC)])

        @pl.when(c == 0)
        def _():
            run_half(celo, bvlo, sasblo_o, elo_o, 0)

        @pl.when(c == 1)
        def _():
            run_half(cehi, bvhi, sasbhi_o, ehi_o, H)

    return sck


@functools.cache
def _make_sc_deg_kernel():
    mesh = plsc.VectorSubcoreMesh(core_axis_name="c", subcore_axis_name="s")
    outs = (
        jax.ShapeDtypeStruct((NP, D), _f32),
        jax.ShapeDtypeStruct((NP, D), _f32),
    )
    scratch = [
        pltpu.VMEM((BE,), jnp.int32),
        pltpu.VMEM((BE, D), _f32),
        pltpu.VMEM((RC, D), _f32),
        pltpu.VMEM_SHARED((NP, D), _f32),
    ]

    @functools.partial(pl.kernel, out_type=outs, mesh=mesh,
                       scratch_types=scratch)
    def degk(dst, deg0_o, deg1_o, idxd, ones, stage, deg_sh):
        c = lax.axis_index("c")
        s = lax.axis_index("s")
        row0 = s * RPC

        def fill_ones(r, _):
            for k in range(D // L):
                ones[r, pl.ds(k * L, L)] = jnp.full((L,), 1.0, _f32)
            return _
        lax.fori_loop(0, BE, fill_ones, 0)

        def run(deg_o, e_off):
            _zero_stage(stage, D)
            for j in range(NRC):
                pltpu.sync_copy(stage, deg_sh.at[pl.ds(row0 + j * RC, RC)])
            plsc.subcore_barrier()
            base0 = e_off + s * EPC2

            def blk(t, _):
                base = base0 + t * BE
                pltpu.sync_copy(dst.at[pl.ds(base, BE)], idxd)
                pltpu.sync_copy(ones, deg_sh.at[idxd], add=True)
                return _
            lax.fori_loop(0, NB2, blk, 0)
            plsc.subcore_barrier()

            for j in range(NRC):
                r0 = row0 + j * RC
                pltpu.sync_copy(deg_sh.at[pl.ds(r0, RC)], stage)
                pltpu.sync_copy(stage, deg_o.at[pl.ds(r0, RC)])

        @pl.when(c == 0)
        def _():
            run(deg0_o, 0)

        @pl.when(c == 1)
        def _():
            run(deg1_o, N_EDGES // 2)

    return degk


def _sc_edge(*args):
    return _make_sc_edge_kernel()(*args)


def _sc_deg(*args):
    return _make_sc_deg_kernel()(*args)


def kernel(x, edge_w, edge_index, batch, params):
    del batch
    src = edge_index[0].astype(jnp.int32)
    dst = edge_index[1].astype(jnp.int32)

    p = params

    def wt(w):
        return w.T

    def bb(b):
        return b.reshape(1, D)

    deg0, deg1 = _sc_deg(dst)

    lp = p["layers"][0]
    uh, ah, bvlo, bvhi = _node_first(
        x, wt(p["emb_h_w"]), bb(p["emb_h_b"]),
        wt(lp["U_w"]), bb(lp["U_b"]), wt(lp["V_w"]), bb(lp["V_b"]),
        wt(lp["A_w"]), bb(lp["A_b"]), wt(lp["B_w"]), bb(lp["B_b"]))
    celo, cehi = _edge_first(
        edge_w, p["emb_e_w"].reshape(1, D), bb(p["emb_e_b"]),
        wt(lp["C_w"]), bb(lp["C_b"]))
    sasblo, sasbhi, elo, ehi = _sc_edge(
        celo, cehi, ah, bvlo, bvhi, src, dst)

    for li in range(1, len(p["layers"])):
        lp = p["layers"][li]
        uh, ah, bvlo, bvhi = _node_rest(
            uh, sasblo, sasbhi, deg0, deg1,
            wt(lp["U_w"]), bb(lp["U_b"]), wt(lp["V_w"]), bb(lp["V_b"]),
            wt(lp["A_w"]), bb(lp["A_b"]), wt(lp["B_w"]), bb(lp["B_b"]))
        celo, cehi = _edge_rest(elo, ehi, wt(lp["C_w"]), bb(lp["C_b"]))
        sasblo, sasbhi, elo, ehi = _sc_edge(
            celo, cehi, ah, bvlo, bvhi, src, dst)

    return _fin(uh, sasblo, sasbhi, deg0, deg1)


# speedup vs baseline: 3.8191x; 1.1599x over previous
import functools

import jax
import jax.numpy as jnp
from jax import lax
from jax.experimental import pallas as pl
from jax.experimental.pallas import tpu as pltpu
from jax.experimental.pallas import tpu_sc as plsc

N_NODES = 10000
N_EDGES = 320000
D = 128
H = D // 2
NS = 16
L = 16

EPC = N_EDGES // NS
BE = 40
NB = EPC // BE
EPC2 = N_EDGES // (2 * NS)
NB2 = EPC2 // BE
NP = 10240
RPC = NP // NS
RC = 32
NRC = RPC // RC

_EPS = 1e-16


_R = 2000
_RE = 2000


def _h_from_parts(uh, sasb_lo, sasb_hi, deg0, deg1):
    sa = jnp.concatenate([sasb_lo[:, :H], sasb_hi[:, :H]], axis=1)
    sb = jnp.concatenate([sasb_lo[:, H:], sasb_hi[:, H:]], axis=1)
    h = jnp.maximum(uh + sa / (sb + _EPS), 0.0)
    deg = deg0[:, :1] + deg1[:, :1]
    return jnp.where(deg > 0.0, h, 0.0)


def _mm(h, w_ref, b_ref):
    return jnp.dot(h, w_ref[...], preferred_element_type=jnp.float32) + b_ref[...]


def _emit_node_outs(h, vw, vb, aw, ab, bw, bb, ah_o, bvlo_o, bvhi_o):
    vh = _mm(h, vw, vb)
    ah = _mm(h, aw, ab)
    bh = _mm(h, bw, bb)
    ah_o[...] = ah
    bvlo_o[...] = jnp.concatenate([bh[:, :H], vh[:, :H]], axis=1)
    bvhi_o[...] = jnp.concatenate([bh[:, H:], vh[:, H:]], axis=1)


def _node_first_body(x_ref, ew_ref, eb_ref, uw_ref, ub_ref, vw_ref, vb_ref,
                     aw_ref, ab_ref, bw_ref, bb_ref,
                     uh_o, ah_o, bvlo_o, bvhi_o):
    h = _mm(x_ref[...], ew_ref, eb_ref)
    uh_o[...] = _mm(h, uw_ref, ub_ref)
    _emit_node_outs(h, vw_ref, vb_ref, aw_ref, ab_ref, bw_ref, bb_ref,
                    ah_o, bvlo_o, bvhi_o)


def _node_rest_body(uhp_ref, sasblo_ref, sasbhi_ref, deg0_ref, deg1_ref,
                    uw_ref, ub_ref, vw_ref, vb_ref, aw_ref, ab_ref,
                    bw_ref, bb_ref,
                    uh_o, ah_o, bvlo_o, bvhi_o):
    h = _h_from_parts(uhp_ref[...], sasblo_ref[...], sasbhi_ref[...],
                      deg0_ref[...], deg1_ref[...])
    uh_o[...] = _mm(h, uw_ref, ub_ref)
    _emit_node_outs(h, vw_ref, vb_ref, aw_ref, ab_ref, bw_ref, bb_ref,
                    ah_o, bvlo_o, bvhi_o)


def _edge_first_body(ew_ref, wt_ref, eb_ref, ct_ref, cb_ref, celo_o, cehi_o):
    e0 = ew_ref[...] * wt_ref[...] + eb_ref[...]
    ce = _mm(e0, ct_ref, cb_ref)
    celo_o[...] = ce[:, :H]
    cehi_o[...] = ce[:, H:]


def _edge_rest_body(elo_ref, ehi_ref, ct_ref, cb_ref, celo_o, cehi_o):
    e = jnp.concatenate([elo_ref[...], ehi_ref[...]], axis=1)
    ce = _mm(e, ct_ref, cb_ref)
    celo_o[...] = ce[:, :H]
    cehi_o[...] = ce[:, H:]


def _fin_body(uhp_ref, sasblo_ref, sasbhi_ref, deg0_ref, deg1_ref, out_o):
    i = pl.program_id(0)
    h = _h_from_parts(uhp_ref[...], sasblo_ref[...], sasbhi_ref[...],
                      deg0_ref[...], deg1_ref[...])
    part = jnp.sum(h, axis=0, keepdims=True)

    @pl.when(i == 0)
    def _():
        out_o[...] = part

    @pl.when(i > 0)
    def _():
        out_o[...] = out_o[...] + part

    @pl.when(i == (N_NODES // _R) - 1)
    def _():
        out_o[...] = out_o[...] * (1.0 / N_NODES)


def _full_spec(shape):
    return pl.BlockSpec(shape, lambda i: (0, 0))


def _row_spec(block_rows, cols):
    return pl.BlockSpec((block_rows, cols), lambda i: (i, 0))


_W = _full_spec((D, D))
_B = _full_spec((1, D))

_NODE_OUTS = (
    jax.ShapeDtypeStruct((N_NODES, D), jnp.float32),
    jax.ShapeDtypeStruct((N_NODES, D), jnp.float32),
    jax.ShapeDtypeStruct((N_NODES, D), jnp.float32),
    jax.ShapeDtypeStruct((N_NODES, D), jnp.float32),
)
_NODE_OUT_SPECS = (_row_spec(_R, D),) * 4

_node_first = pl.pallas_call(
    _node_first_body,
    grid=(N_NODES // _R,),
    in_specs=[_row_spec(_R, D), _W, _B, _W, _B, _W, _B, _W, _B, _W, _B],
    out_specs=_NODE_OUT_SPECS,
    out_shape=_NODE_OUTS,
)

_node_rest = pl.pallas_call(
    _node_rest_body,
    grid=(N_NODES // _R,),
    in_specs=[_row_spec(_R, D),
              _row_spec(_R, D), _row_spec(_R, D),
              _row_spec(_R, D), _row_spec(_R, D),
              _W, _B, _W, _B, _W, _B, _W, _B],
    out_specs=_NODE_OUT_SPECS,
    out_shape=_NODE_OUTS,
)

_EDGE_OUTS = (
    jax.ShapeDtypeStruct((N_EDGES, H), jnp.float32),
    jax.ShapeDtypeStruct((N_EDGES, H), jnp.float32),
)

_edge_first = pl.pallas_call(
    _edge_first_body,
    grid=(N_EDGES // _RE,),
    in_specs=[_row_spec(_RE, 1), _full_spec((1, D)), _B, _W, _B],
    out_specs=(_row_spec(_RE, H), _row_spec(_RE, H)),
    out_shape=_EDGE_OUTS,
)

_edge_rest = pl.pallas_call(
    _edge_rest_body,
    grid=(N_EDGES // _RE,),
    in_specs=[_row_spec(_RE, H), _row_spec(_RE, H), _W, _B],
    out_specs=(_row_spec(_RE, H), _row_spec(_RE, H)),
    out_shape=_EDGE_OUTS,
)

_fin = pl.pallas_call(
    _fin_body,
    grid=(N_NODES // _R,),
    in_specs=[_row_spec(_R, D),
              _row_spec(_R, D), _row_spec(_R, D),
              _row_spec(_R, D), _row_spec(_R, D)],
    out_specs=pl.BlockSpec((1, D), lambda i: (0, 0)),
    out_shape=jax.ShapeDtypeStruct((1, D), jnp.float32),
)


_f32 = jnp.float32


def _zero_stage(stage, cols):
    rows = stage.shape[0]

    def zrow(r, _):
        for k in range(cols // L):
            stage[r, pl.ds(k * L, L)] = jnp.zeros((L,), _f32)
        return _
    lax.fori_loop(0, rows, zrow, 0)


@functools.cache
def _make_sc_edge_kernel(write_e: bool):
    mesh = plsc.VectorSubcoreMesh(core_axis_name="c", subcore_axis_name="s")
    outs = [
        jax.ShapeDtypeStruct((NP, D), _f32),
        jax.ShapeDtypeStruct((NP, D), _f32),
    ]
    if write_e:
        outs += [
            jax.ShapeDtypeStruct((N_EDGES, H), _f32),
            jax.ShapeDtypeStruct((N_EDGES, H), _f32),
        ]
    scratch = [
        pltpu.VMEM((BE,), jnp.int32),
        pltpu.VMEM((BE,), jnp.int32),
        pltpu.VMEM((BE,), jnp.int32),
        pltpu.VMEM((BE,), jnp.int32),
        pltpu.VMEM((BE, D), _f32),
        pltpu.VMEM((BE, D), _f32),
        pltpu.VMEM((BE, D), _f32),
        pltpu.VMEM((BE, D), _f32),
        pltpu.VMEM((BE, H), _f32),
        pltpu.VMEM((BE, H), _f32),
        pltpu.VMEM_SHARED((NP, D), _f32),
        pltpu.SemaphoreType.DMA,
        pltpu.SemaphoreType.DMA,
        pltpu.SemaphoreType.DMA,
        pltpu.SemaphoreType.DMA,
        pltpu.SemaphoreType.DMA,
        pltpu.SemaphoreType.DMA,
    ]

    @functools.partial(pl.kernel, out_type=tuple(outs), mesh=mesh,
                       scratch_types=scratch)
    def sck(*refs):
        celo, cehi, ah, bvlo, bvhi, src, dst = refs[:7]
        pos = 7
        sasblo_o, sasbhi_o = refs[pos:pos + 2]
        pos += 2
        if write_e:
            elo_o, ehi_o = refs[pos:pos + 2]
            pos += 2
        else:
            elo_o = ehi_o = None
        (idxs0, idxs1, idxd0, idxd1, ar0, ar1, bvr0, bvr1, cer0, cer1,
         sasb_sh, sA0, sA1, sB0, sB1, sC0, sC1) = refs[pos:pos + 17]
        idxss = (idxs0, idxs1)
        idxds = (idxd0, idxd1)
        ars = (ar0, ar1)
        bvrs = (bvr0, bvr1)
        cers = (cer0, cer1)
        sAs = (sA0, sA1)
        sBs = (sB0, sB1)
        sCs = (sC0, sC1)

        c = lax.axis_index("c")
        s = lax.axis_index("s")
        row0 = s * RPC
        base0 = s * EPC

        def run_half(ce_h, bv_h, sasb_o, e_o, col0):
            _zero_stage(ar0, D)
            for j in range(RPC // BE):
                pltpu.sync_copy(ar0, sasb_sh.at[pl.ds(row0 + j * BE, BE)])
            plsc.subcore_barrier()

            def start(tb, b):
                base = base0 + tb * BE
                pltpu.sync_copy(src.at[pl.ds(base, BE)], idxss[b])
                pltpu.sync_copy(dst.at[pl.ds(base, BE)], idxds[b])
                pltpu.async_copy(ah.at[idxds[b]], ars[b], sAs[b])
                pltpu.async_copy(bv_h.at[idxss[b]], bvrs[b], sBs[b])
                pltpu.async_copy(ce_h.at[pl.ds(base, BE)], cers[b], sCs[b])

            def finish(tb, b):
                base = base0 + tb * BE
                ar, bvr, cer = ars[b], bvrs[b], cers[b]
                pltpu.make_async_copy(ah.at[idxds[b]], ar, sAs[b]).wait()
                pltpu.make_async_copy(bv_h.at[idxss[b]], bvr, sBs[b]).wait()
                pltpu.make_async_copy(
                    ce_h.at[pl.ds(base, BE)], cer, sCs[b]).wait()

                def row(r, _):
                    for k in range(H // L):
                        a_k = ar[r, pl.ds(col0 + k * L, L)]
                        b_k = bvr[r, pl.ds(k * L, L)]
                        v_k = bvr[r, pl.ds(H + k * L, L)]
                        ce_k = cer[r, pl.ds(k * L, L)]
                        eij = a_k + b_k + ce_k
                        sg = 1.0 / (1.0 + jnp.exp(-eij))
                        if write_e:
                            cer[r, pl.ds(k * L, L)] = jnp.maximum(eij, 0.0)
                        ar[r, pl.ds(k * L, L)] = v_k * sg
                        ar[r, pl.ds(H + k * L, L)] = sg
                    return _
                lax.fori_loop(0, BE, row, 0)

                pltpu.sync_copy(ar, sasb_sh.at[idxds[b]], add=True)
                if write_e:
                    pltpu.sync_copy(cer, e_o.at[pl.ds(base, BE)])

            start(0, 0)

            def outer(t, _):
                t2 = 2 * t
                start(t2 + 1, 1)
                finish(t2, 0)
                start(t2 + 2, 0)
                finish(t2 + 1, 1)
                return _
            lax.fori_loop(0, NB // 2 - 1, outer, 0)
            start(NB - 1, 1)
            finish(NB - 2, 0)
            finish(NB - 1, 1)
            plsc.subcore_barrier()

            for j in range(RPC // BE):
                r0 = row0 + j * BE
                pltpu.sync_copy(sasb_sh.at[pl.ds(r0, BE)], ar0)
                pltpu.sync_copy(ar0, sasb_o.at[pl.ds(r0, BE)])

        @pl.when(c == 0)
        def _():
            run_half(celo, bvlo, sasblo_o, elo_o, 0)

        @pl.when(c == 1)
        def _():
            run_half(cehi, bvhi, sasbhi_o, ehi_o, H)

    return sck


@functools.cache
def _make_sc_deg_kernel():
    mesh = plsc.VectorSubcoreMesh(core_axis_name="c", subcore_axis_name="s")
    outs = (
        jax.ShapeDtypeStruct((NP, D), _f32),
        jax.ShapeDtypeStruct((NP, D), _f32),
    )
    scratch = [
        pltpu.VMEM((BE,), jnp.int32),
        pltpu.VMEM((BE, D), _f32),
        pltpu.VMEM((RC, D), _f32),
        pltpu.VMEM_SHARED((NP, D), _f32),
    ]

    @functools.partial(pl.kernel, out_type=outs, mesh=mesh,
                       scratch_types=scratch)
    def degk(dst, deg0_o, deg1_o, idxd, ones, stage, deg_sh):
        c = lax.axis_index("c")
        s = lax.axis_index("s")
        row0 = s * RPC

        def fill_ones(r, _):
            for k in range(D // L):
                ones[r, pl.ds(k * L, L)] = jnp.full((L,), 1.0, _f32)
            return _
        lax.fori_loop(0, BE, fill_ones, 0)

        def run(deg_o, e_off):
            _zero_stage(stage, D)
            for j in range(NRC):
                pltpu.sync_copy(stage, deg_sh.at[pl.ds(row0 + j * RC, RC)])
            plsc.subcore_barrier()
            base0 = e_off + s * EPC2

            def blk(t, _):
                base = base0 + t * BE
                pltpu.sync_copy(dst.at[pl.ds(base, BE)], idxd)
                pltpu.sync_copy(ones, deg_sh.at[idxd], add=True)
                return _
            lax.fori_loop(0, NB2, blk, 0)
            plsc.subcore_barrier()

            for j in range(NRC):
                r0 = row0 + j * RC
                pltpu.sync_copy(deg_sh.at[pl.ds(r0, RC)], stage)
                pltpu.sync_copy(stage, deg_o.at[pl.ds(r0, RC)])

        @pl.when(c == 0)
        def _():
            run(deg0_o, 0)

        @pl.when(c == 1)
        def _():
            run(deg1_o, N_EDGES // 2)

    return degk


def _sc_edge(*args):
    return _make_sc_edge_kernel(True)(*args)


def _sc_edge_last(*args):
    return _make_sc_edge_kernel(False)(*args)


def _sc_deg(*args):
    return _make_sc_deg_kernel()(*args)


def kernel(x, edge_w, edge_index, batch, params):
    del batch
    src = edge_index[0].astype(jnp.int32)
    dst = edge_index[1].astype(jnp.int32)

    p = params

    def wt(w):
        return w.T

    def bb(b):
        return b.reshape(1, D)

    deg0, deg1 = _sc_deg(dst)

    lp = p["layers"][0]
    uh, ah, bvlo, bvhi = _node_first(
        x, wt(p["emb_h_w"]), bb(p["emb_h_b"]),
        wt(lp["U_w"]), bb(lp["U_b"]), wt(lp["V_w"]), bb(lp["V_b"]),
        wt(lp["A_w"]), bb(lp["A_b"]), wt(lp["B_w"]), bb(lp["B_b"]))
    celo, cehi = _edge_first(
        edge_w, p["emb_e_w"].reshape(1, D), bb(p["emb_e_b"]),
        wt(lp["C_w"]), bb(lp["C_b"]))
    sasblo, sasbhi, elo, ehi = _sc_edge(
        celo, cehi, ah, bvlo, bvhi, src, dst)

    n_layers = len(p["layers"])
    for li in range(1, n_layers):
        lp = p["layers"][li]
        uh, ah, bvlo, bvhi = _node_rest(
            uh, sasblo, sasbhi, deg0, deg1,
            wt(lp["U_w"]), bb(lp["U_b"]), wt(lp["V_w"]), bb(lp["V_b"]),
            wt(lp["A_w"]), bb(lp["A_b"]), wt(lp["B_w"]), bb(lp["B_b"]))
        celo, cehi = _edge_rest(elo, ehi, wt(lp["C_w"]), bb(lp["C_b"]))
        if li < n_layers - 1:
            sasblo, sasbhi, elo, ehi = _sc_edge(
                celo, cehi, ah, bvlo, bvhi, src, dst)
        else:
            sasblo, sasbhi = _sc_edge_last(
                celo, cehi, ah, bvlo, bvhi, src, dst)

    return _fin(uh, sasblo, sasbhi, deg0, deg1)


# speedup vs baseline: 4.6187x; 1.2094x over previous
import functools

import jax
import jax.numpy as jnp
from jax import lax
from jax.experimental import pallas as pl
from jax.experimental.pallas import tpu as pltpu
from jax.experimental.pallas import tpu_sc as plsc

N_NODES = 10000
N_EDGES = 320000
D = 128
H = D // 2
NS = 16
L = 16

EPC = N_EDGES // NS
BE = 40
NB = EPC // BE
EPC2 = N_EDGES // (2 * NS)
NB2 = EPC2 // BE
NP = 10240
RPC = NP // NS
RC = 32
NRC = RPC // RC

_EPS = 1e-16


_R = 2000
_RE = 2000


def _h_from_parts(uh, sasb_lo, sasb_hi, deg0, deg1):
    sa = jnp.concatenate([sasb_lo[:, :H], sasb_hi[:, :H]], axis=1)
    sb = jnp.concatenate([sasb_lo[:, H:], sasb_hi[:, H:]], axis=1)
    h = jnp.maximum(uh + sa / (sb + _EPS), 0.0)
    deg = deg0[:, :1] + deg1[:, :1]
    return jnp.where(deg > 0.0, h, 0.0)


def _mm(h, w_ref, b_ref):
    return jnp.dot(h, w_ref[...], preferred_element_type=jnp.float32) + b_ref[...]


def _emit_node_outs(h, vw, vb, aw, ab, bw, bb, ah_o, bvlo_o, bvhi_o):
    vh = _mm(h, vw, vb)
    ah = _mm(h, aw, ab)
    bh = _mm(h, bw, bb)
    ah_o[...] = ah
    bvlo_o[...] = jnp.concatenate([bh[:, :H], vh[:, :H]], axis=1)
    bvhi_o[...] = jnp.concatenate([bh[:, H:], vh[:, H:]], axis=1)


def _node_first_body(x_ref, ew_ref, eb_ref, uw_ref, ub_ref, vw_ref, vb_ref,
                     aw_ref, ab_ref, bw_ref, bb_ref,
                     uh_o, ah_o, bvlo_o, bvhi_o):
    h = _mm(x_ref[...], ew_ref, eb_ref)
    uh_o[...] = _mm(h, uw_ref, ub_ref)
    _emit_node_outs(h, vw_ref, vb_ref, aw_ref, ab_ref, bw_ref, bb_ref,
                    ah_o, bvlo_o, bvhi_o)


def _node_rest_body(uhp_ref, sasblo_ref, sasbhi_ref, deg0_ref, deg1_ref,
                    uw_ref, ub_ref, vw_ref, vb_ref, aw_ref, ab_ref,
                    bw_ref, bb_ref,
                    uh_o, ah_o, bvlo_o, bvhi_o):
    h = _h_from_parts(uhp_ref[...], sasblo_ref[...], sasbhi_ref[...],
                      deg0_ref[...], deg1_ref[...])
    uh_o[...] = _mm(h, uw_ref, ub_ref)
    _emit_node_outs(h, vw_ref, vb_ref, aw_ref, ab_ref, bw_ref, bb_ref,
                    ah_o, bvlo_o, bvhi_o)


def _edge_first_body(ew_ref, wt_ref, eb_ref, ct_ref, cb_ref, celo_o, cehi_o):
    e0 = ew_ref[...] * wt_ref[...] + eb_ref[...]
    ce = _mm(e0, ct_ref, cb_ref)
    celo_o[...] = ce[:, :H]
    cehi_o[...] = ce[:, H:]


def _edge_rest_body(elo_ref, ehi_ref, ct_ref, cb_ref, celo_o, cehi_o):
    e = jnp.concatenate([elo_ref[...], ehi_ref[...]], axis=1)
    ce = _mm(e, ct_ref, cb_ref)
    celo_o[...] = ce[:, :H]
    cehi_o[...] = ce[:, H:]


def _fin_body(uhp_ref, sasblo_ref, sasbhi_ref, deg0_ref, deg1_ref, out_o):
    i = pl.program_id(0)
    h = _h_from_parts(uhp_ref[...], sasblo_ref[...], sasbhi_ref[...],
                      deg0_ref[...], deg1_ref[...])
    part = jnp.sum(h, axis=0, keepdims=True)

    @pl.when(i == 0)
    def _():
        out_o[...] = part

    @pl.when(i > 0)
    def _():
        out_o[...] = out_o[...] + part

    @pl.when(i == (N_NODES // _R) - 1)
    def _():
        out_o[...] = out_o[...] * (1.0 / N_NODES)


def _full_spec(shape):
    return pl.BlockSpec(shape, lambda i: (0, 0))


def _row_spec(block_rows, cols):
    return pl.BlockSpec((block_rows, cols), lambda i: (i, 0))


_W = _full_spec((D, D))
_B = _full_spec((1, D))

_NODE_OUTS = (
    jax.ShapeDtypeStruct((N_NODES, D), jnp.float32),
    jax.ShapeDtypeStruct((N_NODES, D), jnp.float32),
    jax.ShapeDtypeStruct((N_NODES, D), jnp.float32),
    jax.ShapeDtypeStruct((N_NODES, D), jnp.float32),
)
_NODE_OUT_SPECS = (_row_spec(_R, D),) * 4

_node_first = pl.pallas_call(
    _node_first_body,
    grid=(N_NODES // _R,),
    in_specs=[_row_spec(_R, D), _W, _B, _W, _B, _W, _B, _W, _B, _W, _B],
    out_specs=_NODE_OUT_SPECS,
    out_shape=_NODE_OUTS,
)

_node_rest = pl.pallas_call(
    _node_rest_body,
    grid=(N_NODES // _R,),
    in_specs=[_row_spec(_R, D),
              _row_spec(_R, D), _row_spec(_R, D),
              _row_spec(_R, D), _row_spec(_R, D),
              _W, _B, _W, _B, _W, _B, _W, _B],
    out_specs=_NODE_OUT_SPECS,
    out_shape=_NODE_OUTS,
)

_EDGE_OUTS = (
    jax.ShapeDtypeStruct((N_EDGES, H), jnp.float32),
    jax.ShapeDtypeStruct((N_EDGES, H), jnp.float32),
)

_edge_first = pl.pallas_call(
    _edge_first_body,
    grid=(N_EDGES // _RE,),
    in_specs=[_row_spec(_RE, 1), _full_spec((1, D)), _B, _W, _B],
    out_specs=(_row_spec(_RE, H), _row_spec(_RE, H)),
    out_shape=_EDGE_OUTS,
)

_edge_rest = pl.pallas_call(
    _edge_rest_body,
    grid=(N_EDGES // _RE,),
    in_specs=[_row_spec(_RE, H), _row_spec(_RE, H), _W, _B],
    out_specs=(_row_spec(_RE, H), _row_spec(_RE, H)),
    out_shape=_EDGE_OUTS,
)

_fin = pl.pallas_call(
    _fin_body,
    grid=(N_NODES // _R,),
    in_specs=[_row_spec(_R, D),
              _row_spec(_R, D), _row_spec(_R, D),
              _row_spec(_R, D), _row_spec(_R, D)],
    out_specs=pl.BlockSpec((1, D), lambda i: (0, 0)),
    out_shape=jax.ShapeDtypeStruct((1, D), jnp.float32),
)


_f32 = jnp.float32


def _zero_stage(stage, cols):
    rows = stage.shape[0]

    def zrow(r, _):
        for k in range(cols // L):
            stage[r, pl.ds(k * L, L)] = jnp.zeros((L,), _f32)
        return _
    lax.fori_loop(0, rows, zrow, 0)


@functools.cache
def _make_sc_edge_kernel(write_e: bool):
    mesh = plsc.VectorSubcoreMesh(core_axis_name="c", subcore_axis_name="s")
    outs = [
        jax.ShapeDtypeStruct((NP, D), _f32),
        jax.ShapeDtypeStruct((NP, D), _f32),
    ]
    if write_e:
        outs += [
            jax.ShapeDtypeStruct((N_EDGES, H), _f32),
            jax.ShapeDtypeStruct((N_EDGES, H), _f32),
        ]
    scratch = [
        pltpu.VMEM((BE,), jnp.int32),
        pltpu.VMEM((BE,), jnp.int32),
        pltpu.VMEM((BE,), jnp.int32),
        pltpu.VMEM((BE,), jnp.int32),
        pltpu.VMEM((BE, D), _f32),
        pltpu.VMEM((BE, D), _f32),
        pltpu.VMEM((BE, D), _f32),
        pltpu.VMEM((BE, D), _f32),
        pltpu.VMEM((BE, H), _f32),
        pltpu.VMEM((BE, H), _f32),
        pltpu.VMEM_SHARED((NP, D), _f32),
        pltpu.SemaphoreType.DMA,
        pltpu.SemaphoreType.DMA,
        pltpu.SemaphoreType.DMA,
        pltpu.SemaphoreType.DMA,
        pltpu.SemaphoreType.DMA,
        pltpu.SemaphoreType.DMA,
    ]

    @functools.partial(pl.kernel, out_type=tuple(outs), mesh=mesh,
                       scratch_types=scratch)
    def sck(*refs):
        celo, cehi, ah, bvlo, bvhi, src, dst = refs[:7]
        pos = 7
        sasblo_o, sasbhi_o = refs[pos:pos + 2]
        pos += 2
        if write_e:
            elo_o, ehi_o = refs[pos:pos + 2]
            pos += 2
        else:
            elo_o = ehi_o = None
        (idxs0, idxs1, idxd0, idxd1, ar0, ar1, bvr0, bvr1, cer0, cer1,
         sasb_sh, sA0, sA1, sB0, sB1, sC0, sC1) = refs[pos:pos + 17]
        idxss = (idxs0, idxs1)
        idxds = (idxd0, idxd1)
        ars = (ar0, ar1)
        bvrs = (bvr0, bvr1)
        cers = (cer0, cer1)
        sAs = (sA0, sA1)
        sBs = (sB0, sB1)
        sCs = (sC0, sC1)

        c = lax.axis_index("c")
        s = lax.axis_index("s")
        row0 = s * RPC
        base0 = s * EPC

        def run_half(ce_h, bv_h, sasb_o, e_o, col0):
            _zero_stage(ar0, D)
            for j in range(RPC // BE):
                pltpu.sync_copy(ar0, sasb_sh.at[pl.ds(row0 + j * BE, BE)])
            plsc.subcore_barrier()

            def start(tb, b):
                base = base0 + tb * BE
                pltpu.sync_copy(src.at[pl.ds(base, BE)], idxss[b])
                pltpu.sync_copy(dst.at[pl.ds(base, BE)], idxds[b])
                pltpu.async_copy(ah.at[idxds[b]], ars[b], sAs[b])
                pltpu.async_copy(bv_h.at[idxss[b]], bvrs[b], sBs[b])
                pltpu.async_copy(ce_h.at[pl.ds(base, BE)], cers[b], sCs[b])

            def finish(tb, b):
                base = base0 + tb * BE
                ar, bvr, cer = ars[b], bvrs[b], cers[b]
                pltpu.make_async_copy(ah.at[idxds[b]], ar, sAs[b]).wait()
                pltpu.make_async_copy(bv_h.at[idxss[b]], bvr, sBs[b]).wait()
                pltpu.make_async_copy(
                    ce_h.at[pl.ds(base, BE)], cer, sCs[b]).wait()

                def row(ru, _):
                    for j in range(4):
                        r = ru * 4 + j
                        for k in range(H // L):
                            a_k = ar[r, pl.ds(col0 + k * L, L)]
                            b_k = bvr[r, pl.ds(k * L, L)]
                            v_k = bvr[r, pl.ds(H + k * L, L)]
                            ce_k = cer[r, pl.ds(k * L, L)]
                            eij = a_k + b_k + ce_k
                            sg = 1.0 / (1.0 + jnp.exp(-eij))
                            if write_e:
                                cer[r, pl.ds(k * L, L)] = jnp.maximum(eij, 0.0)
                            ar[r, pl.ds(k * L, L)] = v_k * sg
                            ar[r, pl.ds(H + k * L, L)] = sg
                    return _
                lax.fori_loop(0, BE // 4, row, 0)

                pltpu.sync_copy(ar, sasb_sh.at[idxds[b]], add=True)
                if write_e:
                    pltpu.sync_copy(cer, e_o.at[pl.ds(base, BE)])

            start(0, 0)

            def outer(t, _):
                t2 = 2 * t
                start(t2 + 1, 1)
                finish(t2, 0)
                start(t2 + 2, 0)
                finish(t2 + 1, 1)
                return _
            lax.fori_loop(0, NB // 2 - 1, outer, 0)
            start(NB - 1, 1)
            finish(NB - 2, 0)
            finish(NB - 1, 1)
            plsc.subcore_barrier()

            for j in range(RPC // BE):
                r0 = row0 + j * BE
                pltpu.sync_copy(sasb_sh.at[pl.ds(r0, BE)], ar0)
                pltpu.sync_copy(ar0, sasb_o.at[pl.ds(r0, BE)])

        @pl.when(c == 0)
        def _():
            run_half(celo, bvlo, sasblo_o, elo_o, 0)

        @pl.when(c == 1)
        def _():
            run_half(cehi, bvhi, sasbhi_o, ehi_o, H)

    return sck


@functools.cache
def _make_sc_deg_kernel():
    mesh = plsc.VectorSubcoreMesh(core_axis_name="c", subcore_axis_name="s")
    outs = (
        jax.ShapeDtypeStruct((NP, D), _f32),
        jax.ShapeDtypeStruct((NP, D), _f32),
    )
    scratch = [
        pltpu.VMEM((BE,), jnp.int32),
        pltpu.VMEM((BE, D), _f32),
        pltpu.VMEM((RC, D), _f32),
        pltpu.VMEM_SHARED((NP, D), _f32),
    ]

    @functools.partial(pl.kernel, out_type=outs, mesh=mesh,
                       scratch_types=scratch)
    def degk(dst, deg0_o, deg1_o, idxd, ones, stage, deg_sh):
        c = lax.axis_index("c")
        s = lax.axis_index("s")
        row0 = s * RPC

        def fill_ones(r, _):
            for k in range(D // L):
                ones[r, pl.ds(k * L, L)] = jnp.full((L,), 1.0, _f32)
            return _
        lax.fori_loop(0, BE, fill_ones, 0)

        def run(deg_o, e_off):
            _zero_stage(stage, D)
            for j in range(NRC):
                pltpu.sync_copy(stage, deg_sh.at[pl.ds(row0 + j * RC, RC)])
            plsc.subcore_barrier()
            base0 = e_off + s * EPC2

            def blk(t, _):
                base = base0 + t * BE
                pltpu.sync_copy(dst.at[pl.ds(base, BE)], idxd)
                pltpu.sync_copy(ones, deg_sh.at[idxd], add=True)
                return _
            lax.fori_loop(0, NB2, blk, 0)
            plsc.subcore_barrier()

            for j in range(NRC):
                r0 = row0 + j * RC
                pltpu.sync_copy(deg_sh.at[pl.ds(r0, RC)], stage)
                pltpu.sync_copy(stage, deg_o.at[pl.ds(r0, RC)])

        @pl.when(c == 0)
        def _():
            run(deg0_o, 0)

        @pl.when(c == 1)
        def _():
            run(deg1_o, N_EDGES // 2)

    return degk


def _sc_edge(*args):
    return _make_sc_edge_kernel(True)(*args)


def _sc_edge_last(*args):
    return _make_sc_edge_kernel(False)(*args)


def _sc_deg(*args):
    return _make_sc_deg_kernel()(*args)


def kernel(x, edge_w, edge_index, batch, params):
    del batch
    src = edge_index[0].astype(jnp.int32)
    dst = edge_index[1].astype(jnp.int32)

    p = params

    def wt(w):
        return w.T

    def bb(b):
        return b.reshape(1, D)

    deg0, deg1 = _sc_deg(dst)

    lp = p["layers"][0]
    uh, ah, bvlo, bvhi = _node_first(
        x, wt(p["emb_h_w"]), bb(p["emb_h_b"]),
        wt(lp["U_w"]), bb(lp["U_b"]), wt(lp["V_w"]), bb(lp["V_b"]),
        wt(lp["A_w"]), bb(lp["A_b"]), wt(lp["B_w"]), bb(lp["B_b"]))
    celo, cehi = _edge_first(
        edge_w, p["emb_e_w"].reshape(1, D), bb(p["emb_e_b"]),
        wt(lp["C_w"]), bb(lp["C_b"]))
    sasblo, sasbhi, elo, ehi = _sc_edge(
        celo, cehi, ah, bvlo, bvhi, src, dst)

    n_layers = len(p["layers"])
    for li in range(1, n_layers):
        lp = p["layers"][li]
        uh, ah, bvlo, bvhi = _node_rest(
            uh, sasblo, sasbhi, deg0, deg1,
            wt(lp["U_w"]), bb(lp["U_b"]), wt(lp["V_w"]), bb(lp["V_b"]),
            wt(lp["A_w"]), bb(lp["A_b"]), wt(lp["B_w"]), bb(lp["B_b"]))
        celo, cehi = _edge_rest(elo, ehi, wt(lp["C_w"]), bb(lp["C_b"]))
        if li < n_layers - 1:
            sasblo, sasbhi, elo, ehi = _sc_edge(
                celo, cehi, ah, bvlo, bvhi, src, dst)
        else:
            sasblo, sasbhi = _sc_edge_last(
                celo, cehi, ah, bvlo, bvhi, src, dst)

    return _fin(uh, sasblo, sasbhi, deg0, deg1)


# speedup vs baseline: 4.9510x; 1.0720x over previous
import functools

import jax
import jax.numpy as jnp
from jax import lax
from jax.experimental import pallas as pl
from jax.experimental.pallas import tpu as pltpu
from jax.experimental.pallas import tpu_sc as plsc

N_NODES = 10000
N_EDGES = 320000
D = 128
H = D // 2
NS = 16
L = 16

EPC = N_EDGES // NS
BE = 40
NB = EPC // BE
EPC2 = N_EDGES // (2 * NS)
NB2 = EPC2 // BE
NP = 10240
RPC = NP // NS
RC = 32
NRC = RPC // RC

_EPS = 1e-16


_R = 2000
_RE = 2000


def _h_from_parts(uh, sasb_lo, sasb_hi, deg0, deg1):
    sa = jnp.concatenate([sasb_lo[:, :H], sasb_hi[:, :H]], axis=1)
    sb = jnp.concatenate([sasb_lo[:, H:], sasb_hi[:, H:]], axis=1)
    h = jnp.maximum(uh + sa / (sb + _EPS), 0.0)
    deg = deg0[:, :1] + deg1[:, :1]
    return jnp.where(deg > 0.0, h, 0.0)


def _mm(h, w_ref, b_ref):
    return jnp.dot(h, w_ref[...], preferred_element_type=jnp.float32) + b_ref[...]


def _emit_node_outs(h, vw, vb, aw, ab, bw, bb, ah_o, bvlo_o, bvhi_o):
    vh = _mm(h, vw, vb)
    ah = _mm(h, aw, ab)
    bh = _mm(h, bw, bb)
    ah_o[...] = ah
    bvlo_o[...] = jnp.concatenate([bh[:, :H], vh[:, :H]], axis=1)
    bvhi_o[...] = jnp.concatenate([bh[:, H:], vh[:, H:]], axis=1)


def _node_first_body(x_ref, ew_ref, eb_ref, uw_ref, ub_ref, vw_ref, vb_ref,
                     aw_ref, ab_ref, bw_ref, bb_ref,
                     uh_o, ah_o, bvlo_o, bvhi_o):
    h = _mm(x_ref[...], ew_ref, eb_ref)
    uh_o[...] = _mm(h, uw_ref, ub_ref)
    _emit_node_outs(h, vw_ref, vb_ref, aw_ref, ab_ref, bw_ref, bb_ref,
                    ah_o, bvlo_o, bvhi_o)


def _node_rest_body(uhp_ref, sasblo_ref, sasbhi_ref, deg0_ref, deg1_ref,
                    uw_ref, ub_ref, vw_ref, vb_ref, aw_ref, ab_ref,
                    bw_ref, bb_ref,
                    uh_o, ah_o, bvlo_o, bvhi_o):
    h = _h_from_parts(uhp_ref[...], sasblo_ref[...], sasbhi_ref[...],
                      deg0_ref[...], deg1_ref[...])
    uh_o[...] = _mm(h, uw_ref, ub_ref)
    _emit_node_outs(h, vw_ref, vb_ref, aw_ref, ab_ref, bw_ref, bb_ref,
                    ah_o, bvlo_o, bvhi_o)


def _edge_first_body(ew_ref, wt_ref, eb_ref, ct_ref, cb_ref, celo_o, cehi_o):
    e0 = ew_ref[...] * wt_ref[...] + eb_ref[...]
    ce = _mm(e0, ct_ref, cb_ref)
    celo_o[...] = ce[:, :H]
    cehi_o[...] = ce[:, H:]


def _edge_rest_body(elo_ref, ehi_ref, ct_ref, cb_ref, celo_o, cehi_o):
    e = jnp.concatenate([elo_ref[...], ehi_ref[...]], axis=1)
    ce = _mm(e, ct_ref, cb_ref)
    celo_o[...] = ce[:, :H]
    cehi_o[...] = ce[:, H:]


def _fin_body(uhp_ref, sasblo_ref, sasbhi_ref, deg0_ref, deg1_ref, out_o):
    i = pl.program_id(0)
    h = _h_from_parts(uhp_ref[...], sasblo_ref[...], sasbhi_ref[...],
                      deg0_ref[...], deg1_ref[...])
    part = jnp.sum(h, axis=0, keepdims=True)

    @pl.when(i == 0)
    def _():
        out_o[...] = part

    @pl.when(i > 0)
    def _():
        out_o[...] = out_o[...] + part

    @pl.when(i == (N_NODES // _R) - 1)
    def _():
        out_o[...] = out_o[...] * (1.0 / N_NODES)


def _full_spec(shape):
    return pl.BlockSpec(shape, lambda i: (0, 0))


def _row_spec(block_rows, cols):
    return pl.BlockSpec((block_rows, cols), lambda i: (i, 0))


_W = _full_spec((D, D))
_B = _full_spec((1, D))

_NODE_OUTS = (
    jax.ShapeDtypeStruct((N_NODES, D), jnp.float32),
    jax.ShapeDtypeStruct((N_NODES, D), jnp.float32),
    jax.ShapeDtypeStruct((N_NODES, D), jnp.float32),
    jax.ShapeDtypeStruct((N_NODES, D), jnp.float32),
)
_NODE_OUT_SPECS = (_row_spec(_R, D),) * 4

_node_first = pl.pallas_call(
    _node_first_body,
    grid=(N_NODES // _R,),
    in_specs=[_row_spec(_R, D), _W, _B, _W, _B, _W, _B, _W, _B, _W, _B],
    out_specs=_NODE_OUT_SPECS,
    out_shape=_NODE_OUTS,
)

_node_rest = pl.pallas_call(
    _node_rest_body,
    grid=(N_NODES // _R,),
    in_specs=[_row_spec(_R, D),
              _row_spec(_R, D), _row_spec(_R, D),
              _row_spec(_R, D), _row_spec(_R, D),
              _W, _B, _W, _B, _W, _B, _W, _B],
    out_specs=_NODE_OUT_SPECS,
    out_shape=_NODE_OUTS,
)

_EDGE_OUTS = (
    jax.ShapeDtypeStruct((N_EDGES, H), jnp.float32),
    jax.ShapeDtypeStruct((N_EDGES, H), jnp.float32),
)

_edge_first = pl.pallas_call(
    _edge_first_body,
    grid=(N_EDGES // _RE,),
    in_specs=[_row_spec(_RE, 1), _full_spec((1, D)), _B, _W, _B],
    out_specs=(_row_spec(_RE, H), _row_spec(_RE, H)),
    out_shape=_EDGE_OUTS,
)

_edge_rest = pl.pallas_call(
    _edge_rest_body,
    grid=(N_EDGES // _RE,),
    in_specs=[_row_spec(_RE, H), _row_spec(_RE, H), _W, _B],
    out_specs=(_row_spec(_RE, H), _row_spec(_RE, H)),
    out_shape=_EDGE_OUTS,
)

_fin = pl.pallas_call(
    _fin_body,
    grid=(N_NODES // _R,),
    in_specs=[_row_spec(_R, D),
              _row_spec(_R, D), _row_spec(_R, D),
              _row_spec(_R, D), _row_spec(_R, D)],
    out_specs=pl.BlockSpec((1, D), lambda i: (0, 0)),
    out_shape=jax.ShapeDtypeStruct((1, D), jnp.float32),
)


_f32 = jnp.float32


def _zero_stage(stage, cols):
    rows = stage.shape[0]

    def zrow(r, _):
        for k in range(cols // L):
            stage[r, pl.ds(k * L, L)] = jnp.zeros((L,), _f32)
        return _
    lax.fori_loop(0, rows, zrow, 0)


@functools.cache
def _make_sc_edge_kernel(write_e: bool):
    mesh = plsc.VectorSubcoreMesh(core_axis_name="c", subcore_axis_name="s")
    outs = [
        jax.ShapeDtypeStruct((NP, D), _f32),
        jax.ShapeDtypeStruct((NP, D), _f32),
    ]
    if write_e:
        outs += [
            jax.ShapeDtypeStruct((N_EDGES, H), _f32),
            jax.ShapeDtypeStruct((N_EDGES, H), _f32),
        ]
    scratch = (
        [pltpu.VMEM((BE,), jnp.int32) for _ in range(4)]
        + [pltpu.VMEM((BE, D), _f32) for _ in range(2)]
        + [pltpu.VMEM((BE, D), _f32) for _ in range(2)]
        + [pltpu.VMEM((BE, H), _f32) for _ in range(2)]
        + [pltpu.VMEM((BE, D), _f32) for _ in range(2)]
        + [pltpu.VMEM_SHARED((NP, D), _f32)]
        + [pltpu.SemaphoreType.DMA] * 10
    )

    @functools.partial(pl.kernel, out_type=tuple(outs), mesh=mesh,
                       scratch_types=scratch)
    def sck(*refs):
        celo, cehi, ah, bvlo, bvhi, src, dst = refs[:7]
        pos = 7
        sasblo_o, sasbhi_o = refs[pos:pos + 2]
        pos += 2
        if write_e:
            elo_o, ehi_o = refs[pos:pos + 2]
            pos += 2
        else:
            elo_o = ehi_o = None
        (idxs0, idxs1, idxd0, idxd1, ar0, ar1, bvr0, bvr1, cer0, cer1,
         scb0, scb1, sasb_sh,
         sA0, sA1, sB0, sB1, sC0, sC1, sS0, sS1, sE0, sE1) = refs[pos:]
        idxss = (idxs0, idxs1)
        idxds = (idxd0, idxd1)
        ars = (ar0, ar1)
        bvrs = (bvr0, bvr1)
        cers = (cer0, cer1)
        scbs = (scb0, scb1)
        sAs = (sA0, sA1)
        sBs = (sB0, sB1)
        sCs = (sC0, sC1)
        sSs = (sS0, sS1)
        sEs = (sE0, sE1)

        c = lax.axis_index("c")
        s = lax.axis_index("s")
        row0 = s * RPC
        base0 = s * EPC

        def run_half(ce_h, bv_h, sasb_o, e_o, col0):
            _zero_stage(ar0, D)
            for j in range(RPC // BE):
                pltpu.sync_copy(ar0, sasb_sh.at[pl.ds(row0 + j * BE, BE)])
            plsc.subcore_barrier()

            def start(tb, b, drain_e):
                base = base0 + tb * BE
                if write_e and drain_e:
                    pltpu.make_async_copy(
                        cers[b], e_o.at[pl.ds(base0, BE)], sEs[b]).wait()
                pltpu.sync_copy(src.at[pl.ds(base, BE)], idxss[b])
                pltpu.sync_copy(dst.at[pl.ds(base, BE)], idxds[b])
                pltpu.async_copy(ah.at[idxds[b]], ars[b], sAs[b])
                pltpu.async_copy(bv_h.at[idxss[b]], bvrs[b], sBs[b])
                pltpu.async_copy(ce_h.at[pl.ds(base, BE)], cers[b], sCs[b])

            def drain_scatter(b):
                pltpu.make_async_copy(
                    scbs[b], sasb_sh.at[idxds[b]], sSs[b]).wait()

            def finish(tb, b, drain):
                base = base0 + tb * BE
                ar, bvr, cer = ars[b], bvrs[b], cers[b]
                scb = scbs[b]
                if drain:
                    drain_scatter(b)
                pltpu.make_async_copy(ah.at[idxds[b]], ar, sAs[b]).wait()
                pltpu.make_async_copy(bv_h.at[idxss[b]], bvr, sBs[b]).wait()
                pltpu.make_async_copy(
                    ce_h.at[pl.ds(base, BE)], cer, sCs[b]).wait()

                def row(ru, _):
                    for j in range(4):
                        r = ru * 4 + j
                        for k in range(H // L):
                            a_k = ar[r, pl.ds(col0 + k * L, L)]
                            b_k = bvr[r, pl.ds(k * L, L)]
                            v_k = bvr[r, pl.ds(H + k * L, L)]
                            ce_k = cer[r, pl.ds(k * L, L)]
                            eij = a_k + b_k + ce_k
                            sg = 1.0 / (1.0 + jnp.exp(-eij))
                            if write_e:
                                cer[r, pl.ds(k * L, L)] = jnp.maximum(eij, 0.0)
                            scb[r, pl.ds(k * L, L)] = v_k * sg
                            scb[r, pl.ds(H + k * L, L)] = sg
                    return _
                lax.fori_loop(0, BE // 4, row, 0)

                pltpu.async_copy(scb, sasb_sh.at[idxds[b]], sSs[b], add=True)
                if write_e:
                    pltpu.async_copy(cer, e_o.at[pl.ds(base, BE)], sEs[b])

            start(0, 0, False)
            start(1, 1, False)
            finish(0, 0, False)
            start(2, 0, True)
            finish(1, 1, False)
            start(3, 1, True)

            def outer(t, _):
                t2 = 2 * t
                finish(t2, 0, True)
                start(t2 + 2, 0, True)
                finish(t2 + 1, 1, True)
                start(t2 + 3, 1, True)
                return _
            lax.fori_loop(1, NB // 2 - 1, outer, 0)
            finish(NB - 2, 0, True)
            finish(NB - 1, 1, True)
            drain_scatter(0)
            drain_scatter(1)
            if write_e:
                pltpu.make_async_copy(
                    cers[0], e_o.at[pl.ds(base0, BE)], sEs[0]).wait()
                pltpu.make_async_copy(
                    cers[1], e_o.at[pl.ds(base0, BE)], sEs[1]).wait()
            plsc.subcore_barrier()

            for j in range(RPC // BE):
                r0 = row0 + j * BE
                pltpu.sync_copy(sasb_sh.at[pl.ds(r0, BE)], ar0)
                pltpu.sync_copy(ar0, sasb_o.at[pl.ds(r0, BE)])

        @pl.when(c == 0)
        def _():
            run_half(celo, bvlo, sasblo_o, elo_o, 0)

        @pl.when(c == 1)
        def _():
            run_half(cehi, bvhi, sasbhi_o, ehi_o, H)

    return sck


@functools.cache
def _make_sc_deg_kernel():
    mesh = plsc.VectorSubcoreMesh(core_axis_name="c", subcore_axis_name="s")
    outs = (
        jax.ShapeDtypeStruct((NP, D), _f32),
        jax.ShapeDtypeStruct((NP, D), _f32),
    )
    scratch = [
        pltpu.VMEM((BE,), jnp.int32),
        pltpu.VMEM((BE, D), _f32),
        pltpu.VMEM((RC, D), _f32),
        pltpu.VMEM_SHARED((NP, D), _f32),
    ]

    @functools.partial(pl.kernel, out_type=outs, mesh=mesh,
                       scratch_types=scratch)
    def degk(dst, deg0_o, deg1_o, idxd, ones, stage, deg_sh):
        c = lax.axis_index("c")
        s = lax.axis_index("s")
        row0 = s * RPC

        def fill_ones(r, _):
            for k in range(D // L):
                ones[r, pl.ds(k * L, L)] = jnp.full((L,), 1.0, _f32)
            return _
        lax.fori_loop(0, BE, fill_ones, 0)

        def run(deg_o, e_off):
            _zero_stage(stage, D)
            for j in range(NRC):
                pltpu.sync_copy(stage, deg_sh.at[pl.ds(row0 + j * RC, RC)])
            plsc.subcore_barrier()
            base0 = e_off + s * EPC2

            def blk(t, _):
                base = base0 + t * BE
                pltpu.sync_copy(dst.at[pl.ds(base, BE)], idxd)
                pltpu.sync_copy(ones, deg_sh.at[idxd], add=True)
                return _
            lax.fori_loop(0, NB2, blk, 0)
            plsc.subcore_barrier()

            for j in range(NRC):
                r0 = row0 + j * RC
                pltpu.sync_copy(deg_sh.at[pl.ds(r0, RC)], stage)
                pltpu.sync_copy(stage, deg_o.at[pl.ds(r0, RC)])

        @pl.when(c == 0)
        def _():
            run(deg0_o, 0)

        @pl.when(c == 1)
        def _():
            run(deg1_o, N_EDGES // 2)

    return degk


def _sc_edge(*args):
    return _make_sc_edge_kernel(True)(*args)


def _sc_edge_last(*args):
    return _make_sc_edge_kernel(False)(*args)


def _sc_deg(*args):
    return _make_sc_deg_kernel()(*args)


def kernel(x, edge_w, edge_index, batch, params):
    del batch
    src = edge_index[0].astype(jnp.int32)
    dst = edge_index[1].astype(jnp.int32)

    p = params

    def wt(w):
        return w.T

    def bb(b):
        return b.reshape(1, D)

    deg0, deg1 = _sc_deg(dst)

    lp = p["layers"][0]
    uh, ah, bvlo, bvhi = _node_first(
        x, wt(p["emb_h_w"]), bb(p["emb_h_b"]),
        wt(lp["U_w"]), bb(lp["U_b"]), wt(lp["V_w"]), bb(lp["V_b"]),
        wt(lp["A_w"]), bb(lp["A_b"]), wt(lp["B_w"]), bb(lp["B_b"]))
    celo, cehi = _edge_first(
        edge_w, p["emb_e_w"].reshape(1, D), bb(p["emb_e_b"]),
        wt(lp["C_w"]), bb(lp["C_b"]))
    sasblo, sasbhi, elo, ehi = _sc_edge(
        celo, cehi, ah, bvlo, bvhi, src, dst)

    n_layers = len(p["layers"])
    for li in range(1, n_layers):
        lp = p["layers"][li]
        uh, ah, bvlo, bvhi = _node_rest(
            uh, sasblo, sasbhi, deg0, deg1,
            wt(lp["U_w"]), bb(lp["U_b"]), wt(lp["V_w"]), bb(lp["V_b"]),
            wt(lp["A_w"]), bb(lp["A_b"]), wt(lp["B_w"]), bb(lp["B_b"]))
        celo, cehi = _edge_rest(elo, ehi, wt(lp["C_w"]), bb(lp["C_b"]))
        if li < n_layers - 1:
            sasblo, sasbhi, elo, ehi = _sc_edge(
                celo, cehi, ah, bvlo, bvhi, src, dst)
        else:
            sasblo, sasbhi = _sc_edge_last(
                celo, cehi, ah, bvlo, bvhi, src, dst)

    return _fin(uh, sasblo, sasbhi, deg0, deg1)


# speedup vs baseline: 5.2096x; 1.0522x over previous
import functools

import jax
import jax.numpy as jnp
from jax import lax
from jax.experimental import pallas as pl
from jax.experimental.pallas import tpu as pltpu
from jax.experimental.pallas import tpu_sc as plsc

N_NODES = 10000
N_EDGES = 320000
D = 128
H = D // 2
NS = 16
L = 16

EH = N_EDGES // 2
EPC = EH // NS
BE = 40
NB = EPC // BE
EPC2 = N_EDGES // (2 * NS)
NB2 = EPC2 // BE
NP = 10240
RPC = NP // NS
RC = 32
NRC = RPC // RC

_EPS = 1e-16


_R = 2000
_RE = 2000


def _h_from_parts(uh, sasb_lo0, sasb_lo1, sasb_hi0, sasb_hi1, deg0, deg1):
    sasb_lo = sasb_lo0 + sasb_lo1
    sasb_hi = sasb_hi0 + sasb_hi1
    sa = jnp.concatenate([sasb_lo[:, :H], sasb_hi[:, :H]], axis=1)
    sb = jnp.concatenate([sasb_lo[:, H:], sasb_hi[:, H:]], axis=1)
    h = jnp.maximum(uh + sa / (sb + _EPS), 0.0)
    deg = deg0[:, :1] + deg1[:, :1]
    return jnp.where(deg > 0.0, h, 0.0)


def _mm(h, w_ref, b_ref):
    return jnp.dot(h, w_ref[...], preferred_element_type=jnp.float32) + b_ref[...]


def _emit_node_outs(h, vw, vb, aw, ab, bw, bb, ah_o, bvlo_o, bvhi_o):
    vh = _mm(h, vw, vb)
    ah = _mm(h, aw, ab)
    bh = _mm(h, bw, bb)
    ah_o[...] = ah
    bvlo_o[...] = jnp.concatenate([bh[:, :H], vh[:, :H]], axis=1)
    bvhi_o[...] = jnp.concatenate([bh[:, H:], vh[:, H:]], axis=1)


def _node_first_body(x_ref, ew_ref, eb_ref, uw_ref, ub_ref, vw_ref, vb_ref,
                     aw_ref, ab_ref, bw_ref, bb_ref,
                     uh_o, ah_o, bvlo_o, bvhi_o):
    h = _mm(x_ref[...], ew_ref, eb_ref)
    uh_o[...] = _mm(h, uw_ref, ub_ref)
    _emit_node_outs(h, vw_ref, vb_ref, aw_ref, ab_ref, bw_ref, bb_ref,
                    ah_o, bvlo_o, bvhi_o)


def _node_rest_body(uhp_ref, sasblo0_ref, sasblo1_ref, sasbhi0_ref,
                    sasbhi1_ref, deg0_ref, deg1_ref,
                    uw_ref, ub_ref, vw_ref, vb_ref, aw_ref, ab_ref,
                    bw_ref, bb_ref,
                    uh_o, ah_o, bvlo_o, bvhi_o):
    h = _h_from_parts(uhp_ref[...], sasblo0_ref[...], sasblo1_ref[...],
                      sasbhi0_ref[...], sasbhi1_ref[...],
                      deg0_ref[...], deg1_ref[...])
    uh_o[...] = _mm(h, uw_ref, ub_ref)
    _emit_node_outs(h, vw_ref, vb_ref, aw_ref, ab_ref, bw_ref, bb_ref,
                    ah_o, bvlo_o, bvhi_o)


def _edge_first_body(ew_ref, wt_ref, eb_ref, ct_ref, cb_ref, celo_o, cehi_o):
    e0 = ew_ref[...] * wt_ref[...] + eb_ref[...]
    ce = _mm(e0, ct_ref, cb_ref)
    celo_o[...] = ce[:, :H]
    cehi_o[...] = ce[:, H:]


def _edge_rest_body(elo_ref, ehi_ref, ct_ref, cb_ref, celo_o, cehi_o):
    e = jnp.concatenate([elo_ref[...], ehi_ref[...]], axis=1)
    ce = _mm(e, ct_ref, cb_ref)
    celo_o[...] = ce[:, :H]
    cehi_o[...] = ce[:, H:]


def _fin_body(uhp_ref, sasblo0_ref, sasblo1_ref, sasbhi0_ref, sasbhi1_ref,
              deg0_ref, deg1_ref, out_o):
    i = pl.program_id(0)
    h = _h_from_parts(uhp_ref[...], sasblo0_ref[...], sasblo1_ref[...],
                      sasbhi0_ref[...], sasbhi1_ref[...],
                      deg0_ref[...], deg1_ref[...])
    part = jnp.sum(h, axis=0, keepdims=True)

    @pl.when(i == 0)
    def _():
        out_o[...] = part

    @pl.when(i > 0)
    def _():
        out_o[...] = out_o[...] + part

    @pl.when(i == (N_NODES // _R) - 1)
    def _():
        out_o[...] = out_o[...] * (1.0 / N_NODES)


def _full_spec(shape):
    return pl.BlockSpec(shape, lambda i: (0, 0))


def _row_spec(block_rows, cols):
    return pl.BlockSpec((block_rows, cols), lambda i: (i, 0))


_W = _full_spec((D, D))
_B = _full_spec((1, D))

_NODE_OUTS = (
    jax.ShapeDtypeStruct((N_NODES, D), jnp.float32),
    jax.ShapeDtypeStruct((N_NODES, D), jnp.float32),
    jax.ShapeDtypeStruct((N_NODES, D), jnp.float32),
    jax.ShapeDtypeStruct((N_NODES, D), jnp.float32),
)
_NODE_OUT_SPECS = (_row_spec(_R, D),) * 4

_node_first = pl.pallas_call(
    _node_first_body,
    grid=(N_NODES // _R,),
    in_specs=[_row_spec(_R, D), _W, _B, _W, _B, _W, _B, _W, _B, _W, _B],
    out_specs=_NODE_OUT_SPECS,
    out_shape=_NODE_OUTS,
)

_node_rest = pl.pallas_call(
    _node_rest_body,
    grid=(N_NODES // _R,),
    in_specs=[_row_spec(_R, D)] + [_row_spec(_R, D)] * 6
             + [_W, _B, _W, _B, _W, _B, _W, _B],
    out_specs=_NODE_OUT_SPECS,
    out_shape=_NODE_OUTS,
)

_EDGE_OUTS = (
    jax.ShapeDtypeStruct((EH, H), jnp.float32),
    jax.ShapeDtypeStruct((EH, H), jnp.float32),
)

_edge_first = pl.pallas_call(
    _edge_first_body,
    grid=(EH // _RE,),
    in_specs=[_row_spec(_RE, 1), _full_spec((1, D)), _B, _W, _B],
    out_specs=(_row_spec(_RE, H), _row_spec(_RE, H)),
    out_shape=_EDGE_OUTS,
)

_edge_rest = pl.pallas_call(
    _edge_rest_body,
    grid=(EH // _RE,),
    in_specs=[_row_spec(_RE, H), _row_spec(_RE, H), _W, _B],
    out_specs=(_row_spec(_RE, H), _row_spec(_RE, H)),
    out_shape=_EDGE_OUTS,
)

_fin = pl.pallas_call(
    _fin_body,
    grid=(N_NODES // _R,),
    in_specs=[_row_spec(_R, D)] + [_row_spec(_R, D)] * 6,
    out_specs=pl.BlockSpec((1, D), lambda i: (0, 0)),
    out_shape=jax.ShapeDtypeStruct((1, D), jnp.float32),
)


_f32 = jnp.float32


def _zero_stage(stage, cols):
    rows = stage.shape[0]

    def zrow(r, _):
        for k in range(cols // L):
            stage[r, pl.ds(k * L, L)] = jnp.zeros((L,), _f32)
        return _
    lax.fori_loop(0, rows, zrow, 0)


@functools.cache
def _make_sc_edge_kernel(write_e: bool, half: int):
    mesh = plsc.VectorSubcoreMesh(core_axis_name="c", subcore_axis_name="s")
    outs = [
        jax.ShapeDtypeStruct((NP, D), _f32),
        jax.ShapeDtypeStruct((NP, D), _f32),
    ]
    if write_e:
        outs += [
            jax.ShapeDtypeStruct((EH, H), _f32),
            jax.ShapeDtypeStruct((EH, H), _f32),
        ]
    scratch = (
        [pltpu.VMEM((BE,), jnp.int32) for _ in range(4)]
        + [pltpu.VMEM((BE, D), _f32) for _ in range(2)]
        + [pltpu.VMEM((BE, D), _f32) for _ in range(2)]
        + [pltpu.VMEM((BE, H), _f32) for _ in range(2)]
        + [pltpu.VMEM((BE, D), _f32) for _ in range(2)]
        + [pltpu.VMEM_SHARED((NP, D), _f32)]
        + [pltpu.SemaphoreType.DMA] * 10
    )

    @functools.partial(pl.kernel, out_type=tuple(outs), mesh=mesh,
                       scratch_types=scratch)
    def sck(*refs):
        celo, cehi, ah, bvlo, bvhi, src, dst = refs[:7]
        pos = 7
        sasblo_o, sasbhi_o = refs[pos:pos + 2]
        pos += 2
        if write_e:
            elo_o, ehi_o = refs[pos:pos + 2]
            pos += 2
        else:
            elo_o = ehi_o = None
        (idxs0, idxs1, idxd0, idxd1, ar0, ar1, bvr0, bvr1, cer0, cer1,
         scb0, scb1, sasb_sh,
         sA0, sA1, sB0, sB1, sC0, sC1, sS0, sS1, sE0, sE1) = refs[pos:]
        idxss = (idxs0, idxs1)
        idxds = (idxd0, idxd1)
        ars = (ar0, ar1)
        bvrs = (bvr0, bvr1)
        cers = (cer0, cer1)
        scbs = (scb0, scb1)
        sAs = (sA0, sA1)
        sBs = (sB0, sB1)
        sCs = (sC0, sC1)
        sSs = (sS0, sS1)
        sEs = (sE0, sE1)

        c = lax.axis_index("c")
        s = lax.axis_index("s")
        row0 = s * RPC
        base0 = s * EPC
        ibase0 = half * EH + s * EPC

        def run_half(ce_h, bv_h, sasb_o, e_o, col0):
            _zero_stage(ar0, D)
            for j in range(RPC // BE):
                pltpu.sync_copy(ar0, sasb_sh.at[pl.ds(row0 + j * BE, BE)])
            plsc.subcore_barrier()

            def start(tb, b, drain_e):
                base = base0 + tb * BE
                ibase = ibase0 + tb * BE
                if write_e and drain_e:
                    pltpu.make_async_copy(
                        cers[b], e_o.at[pl.ds(base0, BE)], sEs[b]).wait()
                pltpu.sync_copy(src.at[pl.ds(ibase, BE)], idxss[b])
                pltpu.sync_copy(dst.at[pl.ds(ibase, BE)], idxds[b])
                pltpu.async_copy(ah.at[idxds[b]], ars[b], sAs[b])
                pltpu.async_copy(bv_h.at[idxss[b]], bvrs[b], sBs[b])
                pltpu.async_copy(ce_h.at[pl.ds(base, BE)], cers[b], sCs[b])

            def drain_scatter(b):
                pltpu.make_async_copy(
                    scbs[b], sasb_sh.at[idxds[b]], sSs[b]).wait()

            def finish(tb, b, drain):
                base = base0 + tb * BE
                ar, bvr, cer = ars[b], bvrs[b], cers[b]
                scb = scbs[b]
                if drain:
                    drain_scatter(b)
                pltpu.make_async_copy(ah.at[idxds[b]], ar, sAs[b]).wait()
                pltpu.make_async_copy(bv_h.at[idxss[b]], bvr, sBs[b]).wait()
                pltpu.make_async_copy(
                    ce_h.at[pl.ds(base, BE)], cer, sCs[b]).wait()

                def row(ru, _):
                    for j in range(4):
                        r = ru * 4 + j
                        for k in range(H // L):
                            a_k = ar[r, pl.ds(col0 + k * L, L)]
                            b_k = bvr[r, pl.ds(k * L, L)]
                            v_k = bvr[r, pl.ds(H + k * L, L)]
                            ce_k = cer[r, pl.ds(k * L, L)]
                            eij = a_k + b_k + ce_k
                            sg = 1.0 / (1.0 + jnp.exp(-eij))
                            if write_e:
                                cer[r, pl.ds(k * L, L)] = jnp.maximum(eij, 0.0)
                            scb[r, pl.ds(k * L, L)] = v_k * sg
                            scb[r, pl.ds(H + k * L, L)] = sg
                    return _
                lax.fori_loop(0, BE // 4, row, 0)

                pltpu.async_copy(scb, sasb_sh.at[idxds[b]], sSs[b], add=True)
                if write_e:
                    pltpu.async_copy(cer, e_o.at[pl.ds(base, BE)], sEs[b])

            start(0, 0, False)
            start(1, 1, False)
            finish(0, 0, False)
            start(2, 0, True)
            finish(1, 1, False)
            start(3, 1, True)

            def outer(t, _):
                t2 = 2 * t
                finish(t2, 0, True)
                start(t2 + 2, 0, True)
                finish(t2 + 1, 1, True)
                start(t2 + 3, 1, True)
                return _
            lax.fori_loop(1, NB // 2 - 1, outer, 0)
            finish(NB - 2, 0, True)
            finish(NB - 1, 1, True)
            drain_scatter(0)
            drain_scatter(1)
            if write_e:
                pltpu.make_async_copy(
                    cers[0], e_o.at[pl.ds(base0, BE)], sEs[0]).wait()
                pltpu.make_async_copy(
                    cers[1], e_o.at[pl.ds(base0, BE)], sEs[1]).wait()
            plsc.subcore_barrier()

            for j in range(RPC // BE):
                r0 = row0 + j * BE
                pltpu.sync_copy(sasb_sh.at[pl.ds(r0, BE)], ar0)
                pltpu.sync_copy(ar0, sasb_o.at[pl.ds(r0, BE)])

        @pl.when(c == 0)
        def _():
            run_half(celo, bvlo, sasblo_o, elo_o, 0)

        @pl.when(c == 1)
        def _():
            run_half(cehi, bvhi, sasbhi_o, ehi_o, H)

    return sck


@functools.cache
def _make_sc_deg_kernel():
    mesh = plsc.VectorSubcoreMesh(core_axis_name="c", subcore_axis_name="s")
    outs = (
        jax.ShapeDtypeStruct((NP, D), _f32),
        jax.ShapeDtypeStruct((NP, D), _f32),
    )
    scratch = [
        pltpu.VMEM((BE,), jnp.int32),
        pltpu.VMEM((BE, D), _f32),
        pltpu.VMEM((RC, D), _f32),
        pltpu.VMEM_SHARED((NP, D), _f32),
    ]

    @functools.partial(pl.kernel, out_type=outs, mesh=mesh,
                       scratch_types=scratch)
    def degk(dst, deg0_o, deg1_o, idxd, ones, stage, deg_sh):
        c = lax.axis_index("c")
        s = lax.axis_index("s")
        row0 = s * RPC

        def fill_ones(r, _):
            for k in range(D // L):
                ones[r, pl.ds(k * L, L)] = jnp.full((L,), 1.0, _f32)
            return _
        lax.fori_loop(0, BE, fill_ones, 0)

        def run(deg_o, e_off):
            _zero_stage(stage, D)
            for j in range(NRC):
                pltpu.sync_copy(stage, deg_sh.at[pl.ds(row0 + j * RC, RC)])
            plsc.subcore_barrier()
            base0 = e_off + s * EPC2

            def blk(t, _):
                base = base0 + t * BE
                pltpu.sync_copy(dst.at[pl.ds(base, BE)], idxd)
                pltpu.sync_copy(ones, deg_sh.at[idxd], add=True)
                return _
            lax.fori_loop(0, NB2, blk, 0)
            plsc.subcore_barrier()

            for j in range(NRC):
                r0 = row0 + j * RC
                pltpu.sync_copy(deg_sh.at[pl.ds(r0, RC)], stage)
                pltpu.sync_copy(stage, deg_o.at[pl.ds(r0, RC)])

        @pl.when(c == 0)
        def _():
            run(deg0_o, 0)

        @pl.when(c == 1)
        def _():
            run(deg1_o, N_EDGES // 2)

    return degk


def _sc_edge(half, *args):
    return _make_sc_edge_kernel(True, half)(*args)


def _sc_edge_last(half, *args):
    return _make_sc_edge_kernel(False, half)(*args)


def _sc_deg(*args):
    return _make_sc_deg_kernel()(*args)


def kernel(x, edge_w, edge_index, batch, params):
    del batch
    src = edge_index[0].astype(jnp.int32)
    dst = edge_index[1].astype(jnp.int32)

    p = params

    def wt(w):
        return w.T

    def bb(b):
        return b.reshape(1, D)

    deg0, deg1 = _sc_deg(dst)

    lp = p["layers"][0]
    uh, ah, bvlo, bvhi = _node_first(
        x, wt(p["emb_h_w"]), bb(p["emb_h_b"]),
        wt(lp["U_w"]), bb(lp["U_b"]), wt(lp["V_w"]), bb(lp["V_b"]),
        wt(lp["A_w"]), bb(lp["A_b"]), wt(lp["B_w"]), bb(lp["B_b"]))
    ew_t = p["emb_e_w"].reshape(1, D)
    ce = [None, None]
    for half in range(2):
        ce[half] = _edge_first(
            edge_w[half * EH:(half + 1) * EH], ew_t, bb(p["emb_e_b"]),
            wt(lp["C_w"]), bb(lp["C_b"]))
    sasb0 = _sc_edge(0, ce[0][0], ce[0][1], ah, bvlo, bvhi, src, dst)
    sasb1 = _sc_edge(1, ce[1][0], ce[1][1], ah, bvlo, bvhi, src, dst)
    e_lo = [sasb0[2], sasb1[2]]
    e_hi = [sasb0[3], sasb1[3]]
    sasblo0, sasbhi0 = sasb0[0], sasb0[1]
    sasblo1, sasbhi1 = sasb1[0], sasb1[1]

    n_layers = len(p["layers"])
    for li in range(1, n_layers):
        lp = p["layers"][li]
        uh, ah, bvlo, bvhi = _node_rest(
            uh, sasblo0, sasblo1, sasbhi0, sasbhi1, deg0, deg1,
            wt(lp["U_w"]), bb(lp["U_b"]), wt(lp["V_w"]), bb(lp["V_b"]),
            wt(lp["A_w"]), bb(lp["A_b"]), wt(lp["B_w"]), bb(lp["B_b"]))
        for half in range(2):
            ce[half] = _edge_rest(e_lo[half], e_hi[half],
                                  wt(lp["C_w"]), bb(lp["C_b"]))
        if li < n_layers - 1:
            sasb0 = _sc_edge(0, ce[0][0], ce[0][1], ah, bvlo, bvhi, src, dst)
            sasb1 = _sc_edge(1, ce[1][0], ce[1][1], ah, bvlo, bvhi, src, dst)
            e_lo = [sasb0[2], sasb1[2]]
            e_hi = [sasb0[3], sasb1[3]]
        else:
            sasb0 = _sc_edge_last(0, ce[0][0], ce[0][1], ah, bvlo, bvhi,
                                  src, dst)
            sasb1 = _sc_edge_last(1, ce[1][0], ce[1][1], ah, bvlo, bvhi,
                                  src, dst)
        sasblo0, sasbhi0 = sasb0[0], sasb0[1]
        sasblo1, sasbhi1 = sasb1[0], sasb1[1]

    return _fin(uh, sasblo0, sasblo1, sasbhi0, sasbhi1, deg0, deg1)


# speedup vs baseline: 5.2520x; 1.0082x over previous
import functools

import jax
import jax.numpy as jnp
from jax import lax
from jax.experimental import pallas as pl
from jax.experimental.pallas import tpu as pltpu
from jax.experimental.pallas import tpu_sc as plsc

N_NODES = 10000
N_EDGES = 320000
D = 128
H = D // 2
NS = 16
L = 16

EH = N_EDGES // 2
EPC = EH // NS
BE = 40
NB = EPC // BE
EPC2 = N_EDGES // (2 * NS)
NB2 = EPC2 // BE
NP = 10240
RPC = NP // NS
RC = 32
NRC = RPC // RC

_EPS = 1e-16


_R = 2000
_RE = 2000


def _h_from_parts(uh, sasb_lo0, sasb_lo1, sasb_hi0, sasb_hi1, deg0, deg1):
    sasb_lo = sasb_lo0 + sasb_lo1
    sasb_hi = sasb_hi0 + sasb_hi1
    sa = jnp.concatenate([sasb_lo[:, :H], sasb_hi[:, :H]], axis=1)
    sb = jnp.concatenate([sasb_lo[:, H:], sasb_hi[:, H:]], axis=1)
    h = jnp.maximum(uh + sa / (sb + _EPS), 0.0)
    deg = deg0[:, :1] + deg1[:, :1]
    return jnp.where(deg > 0.0, h, 0.0)


def _mm(h, w_ref, b_ref):
    return jnp.dot(h, w_ref[...], preferred_element_type=jnp.float32) + b_ref[...]


def _emit_node_outs(h, vw, vb, aw, ab, bw, bb, ah_o, bvlo_o, bvhi_o):
    vh = _mm(h, vw, vb)
    ah = _mm(h, aw, ab)
    bh = _mm(h, bw, bb)
    ah_o[...] = ah
    bvlo_o[...] = jnp.concatenate([bh[:, :H], vh[:, :H]], axis=1)
    bvhi_o[...] = jnp.concatenate([bh[:, H:], vh[:, H:]], axis=1)


def _node_first_body(x_ref, ew_ref, eb_ref, uw_ref, ub_ref, vw_ref, vb_ref,
                     aw_ref, ab_ref, bw_ref, bb_ref,
                     uh_o, ah_o, bvlo_o, bvhi_o):
    h = _mm(x_ref[...], ew_ref, eb_ref)
    uh_o[...] = _mm(h, uw_ref, ub_ref)
    _emit_node_outs(h, vw_ref, vb_ref, aw_ref, ab_ref, bw_ref, bb_ref,
                    ah_o, bvlo_o, bvhi_o)


def _node_rest_body(uhp_ref, sasblo0_ref, sasblo1_ref, sasbhi0_ref,
                    sasbhi1_ref, deg0_ref, deg1_ref,
                    uw_ref, ub_ref, vw_ref, vb_ref, aw_ref, ab_ref,
                    bw_ref, bb_ref,
                    uh_o, ah_o, bvlo_o, bvhi_o):
    h = _h_from_parts(uhp_ref[...], sasblo0_ref[...], sasblo1_ref[...],
                      sasbhi0_ref[...], sasbhi1_ref[...],
                      deg0_ref[...], deg1_ref[...])
    uh_o[...] = _mm(h, uw_ref, ub_ref)
    _emit_node_outs(h, vw_ref, vb_ref, aw_ref, ab_ref, bw_ref, bb_ref,
                    ah_o, bvlo_o, bvhi_o)


def _edge_first_body(ew_ref, wt_ref, eb_ref, ct_ref, cb_ref, celo_o, cehi_o):
    e0 = ew_ref[...] * wt_ref[...] + eb_ref[...]
    ce = _mm(e0, ct_ref, cb_ref)
    celo_o[...] = ce[:, :H]
    cehi_o[...] = ce[:, H:]


def _edge_rest_body(elo_ref, ehi_ref, ct_ref, cb_ref, celo_o, cehi_o):
    e = jnp.concatenate([elo_ref[...], ehi_ref[...]], axis=1)
    ce = _mm(e, ct_ref, cb_ref)
    celo_o[...] = ce[:, :H]
    cehi_o[...] = ce[:, H:]


def _fin_body(uhp_ref, sasblo0_ref, sasblo1_ref, sasbhi0_ref, sasbhi1_ref,
              deg0_ref, deg1_ref, out_o):
    i = pl.program_id(0)
    h = _h_from_parts(uhp_ref[...], sasblo0_ref[...], sasblo1_ref[...],
                      sasbhi0_ref[...], sasbhi1_ref[...],
                      deg0_ref[...], deg1_ref[...])
    part = jnp.sum(h, axis=0, keepdims=True)

    @pl.when(i == 0)
    def _():
        out_o[...] = part

    @pl.when(i > 0)
    def _():
        out_o[...] = out_o[...] + part

    @pl.when(i == (N_NODES // _R) - 1)
    def _():
        out_o[...] = out_o[...] * (1.0 / N_NODES)


def _full_spec(shape):
    return pl.BlockSpec(shape, lambda i: (0, 0))


def _row_spec(block_rows, cols):
    return pl.BlockSpec((block_rows, cols), lambda i: (i, 0))


_W = _full_spec((D, D))
_B = _full_spec((1, D))

_NODE_OUTS = (
    jax.ShapeDtypeStruct((N_NODES, D), jnp.float32),
    jax.ShapeDtypeStruct((N_NODES, D), jnp.float32),
    jax.ShapeDtypeStruct((N_NODES, D), jnp.float32),
    jax.ShapeDtypeStruct((N_NODES, D), jnp.float32),
)
_NODE_OUT_SPECS = (_row_spec(_R, D),) * 4

_node_first = pl.pallas_call(
    _node_first_body,
    grid=(N_NODES // _R,),
    in_specs=[_row_spec(_R, D), _W, _B, _W, _B, _W, _B, _W, _B, _W, _B],
    out_specs=_NODE_OUT_SPECS,
    out_shape=_NODE_OUTS,
)

_node_rest = pl.pallas_call(
    _node_rest_body,
    grid=(N_NODES // _R,),
    in_specs=[_row_spec(_R, D)] + [_row_spec(_R, D)] * 6
             + [_W, _B, _W, _B, _W, _B, _W, _B],
    out_specs=_NODE_OUT_SPECS,
    out_shape=_NODE_OUTS,
)

_EDGE_OUTS = (
    jax.ShapeDtypeStruct((EH, H), jnp.float32),
    jax.ShapeDtypeStruct((EH, H), jnp.float32),
)

_edge_first = pl.pallas_call(
    _edge_first_body,
    grid=(EH // _RE,),
    in_specs=[_row_spec(_RE, 1), _full_spec((1, D)), _B, _W, _B],
    out_specs=(_row_spec(_RE, H), _row_spec(_RE, H)),
    out_shape=_EDGE_OUTS,
)

_edge_rest = pl.pallas_call(
    _edge_rest_body,
    grid=(EH // _RE,),
    in_specs=[_row_spec(_RE, H), _row_spec(_RE, H), _W, _B],
    out_specs=(_row_spec(_RE, H), _row_spec(_RE, H)),
    out_shape=_EDGE_OUTS,
)

_fin = pl.pallas_call(
    _fin_body,
    grid=(N_NODES // _R,),
    in_specs=[_row_spec(_R, D)] + [_row_spec(_R, D)] * 6,
    out_specs=pl.BlockSpec((1, D), lambda i: (0, 0)),
    out_shape=jax.ShapeDtypeStruct((1, D), jnp.float32),
)


_f32 = jnp.float32


def _zero_stage(stage, cols):
    rows = stage.shape[0]

    def zrow(r, _):
        for k in range(cols // L):
            stage[r, pl.ds(k * L, L)] = jnp.zeros((L,), _f32)
        return _
    lax.fori_loop(0, rows, zrow, 0)


@functools.cache
def _make_sc_edge_kernel(write_e: bool, half: int):
    mesh = plsc.VectorSubcoreMesh(core_axis_name="c", subcore_axis_name="s")
    outs = [
        jax.ShapeDtypeStruct((NP, D), _f32),
        jax.ShapeDtypeStruct((NP, D), _f32),
    ]
    if write_e:
        outs += [
            jax.ShapeDtypeStruct((EH, H), _f32),
            jax.ShapeDtypeStruct((EH, H), _f32),
        ]
    scratch = (
        [pltpu.VMEM((BE,), jnp.int32) for _ in range(4)]
        + [pltpu.VMEM((BE, D), _f32) for _ in range(2)]
        + [pltpu.VMEM((BE, D), _f32) for _ in range(2)]
        + [pltpu.VMEM((BE, H), _f32) for _ in range(2)]
        + [pltpu.VMEM((BE, D), _f32) for _ in range(2)]
        + [pltpu.VMEM_SHARED((NP, D), _f32)]
        + [pltpu.SemaphoreType.DMA] * 10
    )

    @functools.partial(pl.kernel, out_type=tuple(outs), mesh=mesh,
                       scratch_types=scratch)
    def sck(*refs):
        celo, cehi, ah, bvlo, bvhi, src, dst = refs[:7]
        pos = 7
        sasblo_o, sasbhi_o = refs[pos:pos + 2]
        pos += 2
        if write_e:
            elo_o, ehi_o = refs[pos:pos + 2]
            pos += 2
        else:
            elo_o = ehi_o = None
        (idxs0, idxs1, idxd0, idxd1, ar0, ar1, bvr0, bvr1, cer0, cer1,
         scb0, scb1, sasb_sh,
         sA0, sA1, sB0, sB1, sC0, sC1, sS0, sS1, sE0, sE1) = refs[pos:]
        idxss = (idxs0, idxs1)
        idxds = (idxd0, idxd1)
        ars = (ar0, ar1)
        bvrs = (bvr0, bvr1)
        cers = (cer0, cer1)
        scbs = (scb0, scb1)
        sAs = (sA0, sA1)
        sBs = (sB0, sB1)
        sCs = (sC0, sC1)
        sSs = (sS0, sS1)
        sEs = (sE0, sE1)

        c = lax.axis_index("c")
        s = lax.axis_index("s")
        row0 = s * RPC
        base0 = s * EPC
        ibase0 = half * EH + s * EPC

        def run_half(ce_h, bv_h, sasb_o, e_o, col0):
            _zero_stage(ar0, D)
            nch = RPC // BE
            for j in range(nch):
                pltpu.async_copy(
                    ar0, sasb_sh.at[pl.ds(row0 + j * BE, BE)], sA0)
            for j in range(nch):
                pltpu.make_async_copy(
                    ar0, sasb_sh.at[pl.ds(row0, BE)], sA0).wait()
            plsc.subcore_barrier()

            def start(tb, b, drain_e):
                base = base0 + tb * BE
                ibase = ibase0 + tb * BE
                if write_e and drain_e:
                    pltpu.make_async_copy(
                        cers[b], e_o.at[pl.ds(base0, BE)], sEs[b]).wait()
                pltpu.sync_copy(src.at[pl.ds(ibase, BE)], idxss[b])
                pltpu.sync_copy(dst.at[pl.ds(ibase, BE)], idxds[b])
                pltpu.async_copy(ah.at[idxds[b]], ars[b], sAs[b])
                pltpu.async_copy(bv_h.at[idxss[b]], bvrs[b], sBs[b])
                pltpu.async_copy(ce_h.at[pl.ds(base, BE)], cers[b], sCs[b])

            def drain_scatter(b):
                pltpu.make_async_copy(
                    scbs[b], sasb_sh.at[idxds[b]], sSs[b]).wait()

            def finish(tb, b, drain):
                base = base0 + tb * BE
                ar, bvr, cer = ars[b], bvrs[b], cers[b]
                scb = scbs[b]
                if drain:
                    drain_scatter(b)
                pltpu.make_async_copy(ah.at[idxds[b]], ar, sAs[b]).wait()
                pltpu.make_async_copy(bv_h.at[idxss[b]], bvr, sBs[b]).wait()
                pltpu.make_async_copy(
                    ce_h.at[pl.ds(base, BE)], cer, sCs[b]).wait()

                def row(ru, _):
                    for j in range(4):
                        r = ru * 4 + j
                        for k in range(H // L):
                            a_k = ar[r, pl.ds(col0 + k * L, L)]
                            b_k = bvr[r, pl.ds(k * L, L)]
                            v_k = bvr[r, pl.ds(H + k * L, L)]
                            ce_k = cer[r, pl.ds(k * L, L)]
                            eij = a_k + b_k + ce_k
                            sg = 1.0 / (1.0 + jnp.exp(-eij))
                            if write_e:
                                cer[r, pl.ds(k * L, L)] = jnp.maximum(eij, 0.0)
                            scb[r, pl.ds(k * L, L)] = v_k * sg
                            scb[r, pl.ds(H + k * L, L)] = sg
                    return _
                lax.fori_loop(0, BE // 4, row, 0)

                pltpu.async_copy(scb, sasb_sh.at[idxds[b]], sSs[b], add=True)
                if write_e:
                    pltpu.async_copy(cer, e_o.at[pl.ds(base, BE)], sEs[b])

            start(0, 0, False)
            start(1, 1, False)
            finish(0, 0, False)
            start(2, 0, True)
            finish(1, 1, False)
            start(3, 1, True)

            def outer(t, _):
                t2 = 2 * t
                finish(t2, 0, True)
                start(t2 + 2, 0, True)
                finish(t2 + 1, 1, True)
                start(t2 + 3, 1, True)
                return _
            lax.fori_loop(1, NB // 2 - 1, outer, 0)
            finish(NB - 2, 0, True)
            finish(NB - 1, 1, True)
            drain_scatter(0)
            drain_scatter(1)
            if write_e:
                pltpu.make_async_copy(
                    cers[0], e_o.at[pl.ds(base0, BE)], sEs[0]).wait()
                pltpu.make_async_copy(
                    cers[1], e_o.at[pl.ds(base0, BE)], sEs[1]).wait()
            plsc.subcore_barrier()

            ld = sAs
            st = sBs
            pltpu.async_copy(sasb_sh.at[pl.ds(row0, BE)], ars[0], ld[0])
            for j in range(nch):
                b = j % 2
                if j + 1 < nch:
                    nb = (j + 1) % 2
                    if j >= 1:
                        pltpu.make_async_copy(
                            ars[nb], sasb_o.at[pl.ds(row0, BE)],
                            st[nb]).wait()
                    pltpu.async_copy(
                        sasb_sh.at[pl.ds(row0 + (j + 1) * BE, BE)],
                        ars[nb], ld[nb])
                pltpu.make_async_copy(
                    sasb_sh.at[pl.ds(row0, BE)], ars[b], ld[b]).wait()
                pltpu.async_copy(
                    ars[b], sasb_o.at[pl.ds(row0 + j * BE, BE)], st[b])
            pltpu.make_async_copy(
                ars[0], sasb_o.at[pl.ds(row0, BE)], st[0]).wait()
            pltpu.make_async_copy(
                ars[1], sasb_o.at[pl.ds(row0, BE)], st[1]).wait()

        @pl.when(c == 0)
        def _():
            run_half(celo, bvlo, sasblo_o, elo_o, 0)

        @pl.when(c == 1)
        def _():
            run_half(cehi, bvhi, sasbhi_o, ehi_o, H)

    return sck


@functools.cache
def _make_sc_deg_kernel():
    mesh = plsc.VectorSubcoreMesh(core_axis_name="c", subcore_axis_name="s")
    outs = (
        jax.ShapeDtypeStruct((NP, D), _f32),
        jax.ShapeDtypeStruct((NP, D), _f32),
    )
    scratch = [
        pltpu.VMEM((BE,), jnp.int32),
        pltpu.VMEM((BE, D), _f32),
        pltpu.VMEM((RC, D), _f32),
        pltpu.VMEM_SHARED((NP, D), _f32),
    ]

    @functools.partial(pl.kernel, out_type=outs, mesh=mesh,
                       scratch_types=scratch)
    def degk(dst, deg0_o, deg1_o, idxd, ones, stage, deg_sh):
        c = lax.axis_index("c")
        s = lax.axis_index("s")
        row0 = s * RPC

        def fill_ones(r, _):
            for k in range(D // L):
                ones[r, pl.ds(k * L, L)] = jnp.full((L,), 1.0, _f32)
            return _
        lax.fori_loop(0, BE, fill_ones, 0)

        def run(deg_o, e_off):
            _zero_stage(stage, D)
            for j in range(NRC):
                pltpu.sync_copy(stage, deg_sh.at[pl.ds(row0 + j * RC, RC)])
            plsc.subcore_barrier()
            base0 = e_off + s * EPC2

            def blk(t, _):
                base = base0 + t * BE
                pltpu.sync_copy(dst.at[pl.ds(base, BE)], idxd)
                pltpu.sync_copy(ones, deg_sh.at[idxd], add=True)
                return _
            lax.fori_loop(0, NB2, blk, 0)
            plsc.subcore_barrier()

            for j in range(NRC):
                r0 = row0 + j * RC
                pltpu.sync_copy(deg_sh.at[pl.ds(r0, RC)], stage)
                pltpu.sync_copy(stage, deg_o.at[pl.ds(r0, RC)])

        @pl.when(c == 0)
        def _():
            run(deg0_o, 0)

        @pl.when(c == 1)
        def _():
            run(deg1_o, N_EDGES // 2)

    return degk


def _sc_edge(half, *args):
    return _make_sc_edge_kernel(True, half)(*args)


def _sc_edge_last(half, *args):
    return _make_sc_edge_kernel(False, half)(*args)


def _sc_deg(*args):
    return _make_sc_deg_kernel()(*args)


def kernel(x, edge_w, edge_index, batch, params):
    del batch
    src = edge_index[0].astype(jnp.int32)
    dst = edge_index[1].astype(jnp.int32)

    p = params

    def wt(w):
        return w.T

    def bb(b):
        return b.reshape(1, D)

    deg0, deg1 = _sc_deg(dst)

    lp = p["layers"][0]
    uh, ah, bvlo, bvhi = _node_first(
        x, wt(p["emb_h_w"]), bb(p["emb_h_b"]),
        wt(lp["U_w"]), bb(lp["U_b"]), wt(lp["V_w"]), bb(lp["V_b"]),
        wt(lp["A_w"]), bb(lp["A_b"]), wt(lp["B_w"]), bb(lp["B_b"]))
    ew_t = p["emb_e_w"].reshape(1, D)
    ce = [None, None]
    for half in range(2):
        ce[half] = _edge_first(
            edge_w[half * EH:(half + 1) * EH], ew_t, bb(p["emb_e_b"]),
            wt(lp["C_w"]), bb(lp["C_b"]))
    sasb0 = _sc_edge(0, ce[0][0], ce[0][1], ah, bvlo, bvhi, src, dst)
    sasb1 = _sc_edge(1, ce[1][0], ce[1][1], ah, bvlo, bvhi, src, dst)
    e_lo = [sasb0[2], sasb1[2]]
    e_hi = [sasb0[3], sasb1[3]]
    sasblo0, sasbhi0 = sasb0[0], sasb0[1]
    sasblo1, sasbhi1 = sasb1[0], sasb1[1]

    n_layers = len(p["layers"])
    for li in range(1, n_layers):
        lp = p["layers"][li]
        uh, ah, bvlo, bvhi = _node_rest(
            uh, sasblo0, sasblo1, sasbhi0, sasbhi1, deg0, deg1,
            wt(lp["U_w"]), bb(lp["U_b"]), wt(lp["V_w"]), bb(lp["V_b"]),
            wt(lp["A_w"]), bb(lp["A_b"]), wt(lp["B_w"]), bb(lp["B_b"]))
        for half in range(2):
            ce[half] = _edge_rest(e_lo[half], e_hi[half],
                                  wt(lp["C_w"]), bb(lp["C_b"]))
        if li < n_layers - 1:
            sasb0 = _sc_edge(0, ce[0][0], ce[0][1], ah, bvlo, bvhi, src, dst)
            sasb1 = _sc_edge(1, ce[1][0], ce[1][1], ah, bvlo, bvhi, src, dst)
            e_lo = [sasb0[2], sasb1[2]]
            e_hi = [sasb0[3], sasb1[3]]
        else:
            sasb0 = _sc_edge_last(0, ce[0][0], ce[0][1], ah, bvlo, bvhi,
                                  src, dst)
            sasb1 = _sc_edge_last(1, ce[1][0], ce[1][1], ah, bvlo, bvhi,
                                  src, dst)
        sasblo0, sasbhi0 = sasb0[0], sasb0[1]
        sasblo1, sasbhi1 = sasb1[0], sasb1[1]

    return _fin(uh, sasblo0, sasblo1, sasbhi0, sasbhi1, deg0, deg1)


# speedup vs baseline: 5.3294x; 1.0147x over previous
import functools

import jax
import jax.numpy as jnp
from jax import lax
from jax.experimental import pallas as pl
from jax.experimental.pallas import tpu as pltpu
from jax.experimental.pallas import tpu_sc as plsc

N_NODES = 10000
N_EDGES = 320000
D = 128
H = D // 2
NS = 16
L = 16

EH = N_EDGES // 2
EPC = EH // NS
BE = 40
NB = EPC // BE
EPC2 = N_EDGES // (2 * NS)
NB2 = EPC2 // BE
NP = 10240
RPC = NP // NS
RC = 32
NRC = RPC // RC

_EPS = 1e-16


_R = 2000
_RE = 2000


def _h_from_parts(uh, sasb_lo0, sasb_lo1, sasb_hi0, sasb_hi1, deg0, deg1):
    sasb_lo = sasb_lo0 + sasb_lo1
    sasb_hi = sasb_hi0 + sasb_hi1
    sa = jnp.concatenate([sasb_lo[:, :H], sasb_hi[:, :H]], axis=1)
    sb = jnp.concatenate([sasb_lo[:, H:], sasb_hi[:, H:]], axis=1)
    h = jnp.maximum(uh + sa / (sb + _EPS), 0.0)
    deg = deg0[:, :1] + deg1[:, :1]
    return jnp.where(deg > 0.0, h, 0.0)


def _mm(h, w_ref, b_ref):
    return jnp.dot(h, w_ref[...], preferred_element_type=jnp.float32) + b_ref[...]


def _emit_node_outs(h, vw, vb, aw, ab, bw, bb, ah_o, bvlo_o, bvhi_o):
    vh = _mm(h, vw, vb)
    ah = _mm(h, aw, ab)
    bh = _mm(h, bw, bb)
    ah_o[...] = ah
    bvlo_o[...] = jnp.concatenate([bh[:, :H], vh[:, :H]], axis=1)
    bvhi_o[...] = jnp.concatenate([bh[:, H:], vh[:, H:]], axis=1)


def _node_first_body(x_ref, ew_ref, eb_ref, uw_ref, ub_ref, vw_ref, vb_ref,
                     aw_ref, ab_ref, bw_ref, bb_ref,
                     uh_o, ah_o, bvlo_o, bvhi_o):
    h = _mm(x_ref[...], ew_ref, eb_ref)
    uh_o[...] = _mm(h, uw_ref, ub_ref)
    _emit_node_outs(h, vw_ref, vb_ref, aw_ref, ab_ref, bw_ref, bb_ref,
                    ah_o, bvlo_o, bvhi_o)


def _node_rest_body(uhp_ref, sasblo0_ref, sasblo1_ref, sasbhi0_ref,
                    sasbhi1_ref, deg0_ref, deg1_ref,
                    uw_ref, ub_ref, vw_ref, vb_ref, aw_ref, ab_ref,
                    bw_ref, bb_ref,
                    uh_o, ah_o, bvlo_o, bvhi_o):
    h = _h_from_parts(uhp_ref[...], sasblo0_ref[...], sasblo1_ref[...],
                      sasbhi0_ref[...], sasbhi1_ref[...],
                      deg0_ref[...], deg1_ref[...])
    uh_o[...] = _mm(h, uw_ref, ub_ref)
    _emit_node_outs(h, vw_ref, vb_ref, aw_ref, ab_ref, bw_ref, bb_ref,
                    ah_o, bvlo_o, bvhi_o)


def _edge_first_body(ew_ref, wt_ref, eb_ref, ct_ref, cb_ref, celo_o, cehi_o):
    e0 = ew_ref[...] * wt_ref[...] + eb_ref[...]
    ce = _mm(e0, ct_ref, cb_ref)
    celo_o[...] = ce[:, :H]
    cehi_o[...] = ce[:, H:]


def _edge_rest_body(elo_ref, ehi_ref, ct_ref, cb_ref, celo_o, cehi_o):
    e = jnp.concatenate([elo_ref[...], ehi_ref[...]], axis=1)
    ce = _mm(e, ct_ref, cb_ref)
    celo_o[...] = ce[:, :H]
    cehi_o[...] = ce[:, H:]


def _fin_body(uhp_ref, sasblo0_ref, sasblo1_ref, sasbhi0_ref, sasbhi1_ref,
              deg0_ref, deg1_ref, out_o):
    i = pl.program_id(0)
    h = _h_from_parts(uhp_ref[...], sasblo0_ref[...], sasblo1_ref[...],
                      sasbhi0_ref[...], sasbhi1_ref[...],
                      deg0_ref[...], deg1_ref[...])
    part = jnp.sum(h, axis=0, keepdims=True)

    @pl.when(i == 0)
    def _():
        out_o[...] = part

    @pl.when(i > 0)
    def _():
        out_o[...] = out_o[...] + part

    @pl.when(i == (N_NODES // _R) - 1)
    def _():
        out_o[...] = out_o[...] * (1.0 / N_NODES)


def _full_spec(shape):
    return pl.BlockSpec(shape, lambda i: (0, 0))


def _row_spec(block_rows, cols):
    return pl.BlockSpec((block_rows, cols), lambda i: (i, 0))


_W = _full_spec((D, D))
_B = _full_spec((1, D))

_NODE_OUTS = (
    jax.ShapeDtypeStruct((N_NODES, D), jnp.float32),
    jax.ShapeDtypeStruct((N_NODES, D), jnp.float32),
    jax.ShapeDtypeStruct((N_NODES, D), jnp.float32),
    jax.ShapeDtypeStruct((N_NODES, D), jnp.float32),
)
_NODE_OUT_SPECS = (_row_spec(_R, D),) * 4

_node_first = pl.pallas_call(
    _node_first_body,
    grid=(N_NODES // _R,),
    in_specs=[_row_spec(_R, D), _W, _B, _W, _B, _W, _B, _W, _B, _W, _B],
    out_specs=_NODE_OUT_SPECS,
    out_shape=_NODE_OUTS,
)

_node_rest = pl.pallas_call(
    _node_rest_body,
    grid=(N_NODES // _R,),
    in_specs=[_row_spec(_R, D)] + [_row_spec(_R, D)] * 6
             + [_W, _B, _W, _B, _W, _B, _W, _B],
    out_specs=_NODE_OUT_SPECS,
    out_shape=_NODE_OUTS,
)

_EDGE_OUTS = (
    jax.ShapeDtypeStruct((EH, H), jnp.float32),
    jax.ShapeDtypeStruct((EH, H), jnp.float32),
)

_edge_first = pl.pallas_call(
    _edge_first_body,
    grid=(EH // _RE,),
    in_specs=[_row_spec(_RE, 1), _full_spec((1, D)), _B, _W, _B],
    out_specs=(_row_spec(_RE, H), _row_spec(_RE, H)),
    out_shape=_EDGE_OUTS,
)

_edge_rest = pl.pallas_call(
    _edge_rest_body,
    grid=(EH // _RE,),
    in_specs=[_row_spec(_RE, H), _row_spec(_RE, H), _W, _B],
    out_specs=(_row_spec(_RE, H), _row_spec(_RE, H)),
    out_shape=_EDGE_OUTS,
)

_fin = pl.pallas_call(
    _fin_body,
    grid=(N_NODES // _R,),
    in_specs=[_row_spec(_R, D)] + [_row_spec(_R, D)] * 6,
    out_specs=pl.BlockSpec((1, D), lambda i: (0, 0)),
    out_shape=jax.ShapeDtypeStruct((1, D), jnp.float32),
)


_f32 = jnp.float32


def _zero_stage(stage, cols):
    rows = stage.shape[0]

    def zrow(r, _):
        for k in range(cols // L):
            stage[r, pl.ds(k * L, L)] = jnp.zeros((L,), _f32)
        return _
    lax.fori_loop(0, rows, zrow, 0)


@functools.cache
def _make_sc_edge_kernel(write_e: bool, half: int):
    mesh = plsc.VectorSubcoreMesh(core_axis_name="c", subcore_axis_name="s")
    outs = [
        jax.ShapeDtypeStruct((NP, D), _f32),
        jax.ShapeDtypeStruct((NP, D), _f32),
    ]
    if write_e:
        outs += [
            jax.ShapeDtypeStruct((EH, H), _f32),
            jax.ShapeDtypeStruct((EH, H), _f32),
        ]
    scratch = (
        [pltpu.VMEM((BE,), jnp.int32) for _ in range(4)]
        + [pltpu.VMEM((BE, D), _f32) for _ in range(2)]
        + [pltpu.VMEM((BE, D), _f32) for _ in range(2)]
        + [pltpu.VMEM((BE, H), _f32) for _ in range(2)]
        + [pltpu.VMEM((BE, D), _f32) for _ in range(2)]
        + [pltpu.VMEM_SHARED((NP, D), _f32)]
        + [pltpu.SemaphoreType.DMA] * 10
    )

    @functools.partial(pl.kernel, out_type=tuple(outs), mesh=mesh,
                       scratch_types=scratch)
    def sck(*refs):
        celo, cehi, ah, bvlo, bvhi, src, dst = refs[:7]
        pos = 7
        sasblo_o, sasbhi_o = refs[pos:pos + 2]
        pos += 2
        if write_e:
            elo_o, ehi_o = refs[pos:pos + 2]
            pos += 2
        else:
            elo_o = ehi_o = None
        (idxs0, idxs1, idxd0, idxd1, ar0, ar1, bvr0, bvr1, cer0, cer1,
         scb0, scb1, sasb_sh,
         sA0, sA1, sB0, sB1, sC0, sC1, sS0, sS1, sE0, sE1) = refs[pos:]
        idxss = (idxs0, idxs1)
        idxds = (idxd0, idxd1)
        ars = (ar0, ar1)
        bvrs = (bvr0, bvr1)
        cers = (cer0, cer1)
        scbs = (scb0, scb1)
        sAs = (sA0, sA1)
        sBs = (sB0, sB1)
        sCs = (sC0, sC1)
        sSs = (sS0, sS1)
        sEs = (sE0, sE1)

        c = lax.axis_index("c")
        s = lax.axis_index("s")
        row0 = s * RPC
        base0 = s * EPC
        ibase0 = half * EH + s * EPC

        def run_half(ce_h, bv_h, sasb_o, e_o, col0):
            _zero_stage(ar0, D)
            nch = RPC // BE
            for j in range(nch):
                pltpu.async_copy(
                    ar0, sasb_sh.at[pl.ds(row0 + j * BE, BE)], sA0)
            for j in range(nch):
                pltpu.make_async_copy(
                    ar0, sasb_sh.at[pl.ds(row0, BE)], sA0).wait()
            plsc.subcore_barrier()

            def start(tb, b, drain_e):
                base = base0 + tb * BE
                ibase = ibase0 + tb * BE
                if write_e and drain_e:
                    pltpu.make_async_copy(
                        cers[b], e_o.at[pl.ds(base0, BE)], sEs[b]).wait()
                pltpu.sync_copy(src.at[pl.ds(ibase, BE)], idxss[b])
                pltpu.sync_copy(dst.at[pl.ds(ibase, BE)], idxds[b])
                pltpu.async_copy(ah.at[idxds[b]], ars[b], sAs[b])
                pltpu.async_copy(bv_h.at[idxss[b]], bvrs[b], sBs[b])
                pltpu.async_copy(ce_h.at[pl.ds(base, BE)], cers[b], sCs[b])

            def drain_scatter(b):
                pltpu.make_async_copy(
                    scbs[b], sasb_sh.at[idxds[b]], sSs[b]).wait()

            def finish(tb, b, drain):
                base = base0 + tb * BE
                ar, bvr, cer = ars[b], bvrs[b], cers[b]
                scb = scbs[b]
                if drain:
                    drain_scatter(b)
                pltpu.make_async_copy(ah.at[idxds[b]], ar, sAs[b]).wait()
                pltpu.make_async_copy(bv_h.at[idxss[b]], bvr, sBs[b]).wait()
                pltpu.make_async_copy(
                    ce_h.at[pl.ds(base, BE)], cer, sCs[b]).wait()

                def row(ru, _):
                    for j in range(4):
                        r = ru * 4 + j
                        for k in range(H // L):
                            a_k = ar[r, pl.ds(col0 + k * L, L)]
                            b_k = bvr[r, pl.ds(k * L, L)]
                            v_k = bvr[r, pl.ds(H + k * L, L)]
                            ce_k = cer[r, pl.ds(k * L, L)]
                            eij = a_k + b_k + ce_k
                            sg = 1.0 / (1.0 + jnp.exp(-eij))
                            if write_e:
                                cer[r, pl.ds(k * L, L)] = jnp.maximum(eij, 0.0)
                            scb[r, pl.ds(k * L, L)] = v_k * sg
                            scb[r, pl.ds(H + k * L, L)] = sg
                    return _
                lax.fori_loop(0, BE // 4, row, 0)

                pltpu.async_copy(scb, sasb_sh.at[idxds[b]], sSs[b], add=True)
                if write_e:
                    pltpu.async_copy(cer, e_o.at[pl.ds(base, BE)], sEs[b])

            start(0, 0, False)
            start(1, 1, False)
            finish(0, 0, False)
            start(2, 0, True)
            finish(1, 1, False)
            start(3, 1, True)

            def outer(t, _):
                t2 = 2 * t
                finish(t2, 0, True)
                start(t2 + 2, 0, True)
                finish(t2 + 1, 1, True)
                start(t2 + 3, 1, True)
                return _
            lax.fori_loop(1, NB // 2 - 1, outer, 0)
            finish(NB - 2, 0, True)
            finish(NB - 1, 1, True)
            drain_scatter(0)
            drain_scatter(1)
            if write_e:
                pltpu.make_async_copy(
                    cers[0], e_o.at[pl.ds(base0, BE)], sEs[0]).wait()
                pltpu.make_async_copy(
                    cers[1], e_o.at[pl.ds(base0, BE)], sEs[1]).wait()
            plsc.subcore_barrier()

            ld = sAs
            st = sBs
            pltpu.async_copy(sasb_sh.at[pl.ds(row0, BE)], ars[0], ld[0])
            for j in range(nch):
                b = j % 2
                if j + 1 < nch:
                    nb = (j + 1) % 2
                    if j >= 1:
                        pltpu.make_async_copy(
                            ars[nb], sasb_o.at[pl.ds(row0, BE)],
                            st[nb]).wait()
                    pltpu.async_copy(
                        sasb_sh.at[pl.ds(row0 + (j + 1) * BE, BE)],
                        ars[nb], ld[nb])
                pltpu.make_async_copy(
                    sasb_sh.at[pl.ds(row0, BE)], ars[b], ld[b]).wait()
                pltpu.async_copy(
                    ars[b], sasb_o.at[pl.ds(row0 + j * BE, BE)], st[b])
            pltpu.make_async_copy(
                ars[0], sasb_o.at[pl.ds(row0, BE)], st[0]).wait()
            pltpu.make_async_copy(
                ars[1], sasb_o.at[pl.ds(row0, BE)], st[1]).wait()

        @pl.when(c == 0)
        def _():
            run_half(celo, bvlo, sasblo_o, elo_o, 0)

        @pl.when(c == 1)
        def _():
            run_half(cehi, bvhi, sasbhi_o, ehi_o, H)

    return sck


@functools.cache
def _make_sc_deg_kernel():
    mesh = plsc.VectorSubcoreMesh(core_axis_name="c", subcore_axis_name="s")
    outs = (
        jax.ShapeDtypeStruct((NP, D), _f32),
        jax.ShapeDtypeStruct((NP, D), _f32),
    )
    scratch = [
        pltpu.VMEM((BE,), jnp.int32),
        pltpu.VMEM((BE,), jnp.int32),
        pltpu.VMEM((BE, D), _f32),
        pltpu.VMEM((RC, D), _f32),
        pltpu.VMEM_SHARED((NP, D), _f32),
        pltpu.SemaphoreType.DMA,
        pltpu.SemaphoreType.DMA,
    ]

    @functools.partial(pl.kernel, out_type=outs, mesh=mesh,
                       scratch_types=scratch)
    def degk(dst, deg0_o, deg1_o, idxd0, idxd1, ones, stage, deg_sh,
             sS0, sS1):
        idxds = (idxd0, idxd1)
        sSs = (sS0, sS1)
        c = lax.axis_index("c")
        s = lax.axis_index("s")
        row0 = s * RPC

        def fill_ones(r, _):
            for k in range(D // L):
                ones[r, pl.ds(k * L, L)] = jnp.full((L,), 1.0, _f32)
            return _
        lax.fori_loop(0, BE, fill_ones, 0)

        def run(deg_o, e_off):
            _zero_stage(stage, D)
            for j in range(NRC):
                pltpu.sync_copy(stage, deg_sh.at[pl.ds(row0 + j * RC, RC)])
            plsc.subcore_barrier()
            base0 = e_off + s * EPC2

            def load(tb, b):
                pltpu.sync_copy(
                    dst.at[pl.ds(base0 + tb * BE, BE)], idxds[b])

            def scat(b):
                pltpu.async_copy(ones, deg_sh.at[idxds[b]], sSs[b], add=True)

            def drain(b):
                pltpu.make_async_copy(
                    ones, deg_sh.at[idxds[b]], sSs[b]).wait()

            load(0, 0)
            scat(0)
            load(1, 1)
            scat(1)

            def blk(t, _):
                t2 = 2 * t
                drain(0)
                load(t2, 0)
                scat(0)
                drain(1)
                load(t2 + 1, 1)
                scat(1)
                return _
            lax.fori_loop(1, NB2 // 2, blk, 0)
            drain(0)
            drain(1)
            plsc.subcore_barrier()

            for j in range(NRC):
                r0 = row0 + j * RC
                pltpu.sync_copy(deg_sh.at[pl.ds(r0, RC)], stage)
                pltpu.sync_copy(stage, deg_o.at[pl.ds(r0, RC)])

        @pl.when(c == 0)
        def _():
            run(deg0_o, 0)

        @pl.when(c == 1)
        def _():
            run(deg1_o, N_EDGES // 2)

    return degk


def _sc_edge(half, *args):
    return _make_sc_edge_kernel(True, half)(*args)


def _sc_edge_last(half, *args):
    return _make_sc_edge_kernel(False, half)(*args)


def _sc_deg(*args):
    return _make_sc_deg_kernel()(*args)


def kernel(x, edge_w, edge_index, batch, params):
    del batch
    src = edge_index[0].astype(jnp.int32)
    dst = edge_index[1].astype(jnp.int32)

    p = params

    def wt(w):
        return w.T

    def bb(b):
        return b.reshape(1, D)

    deg0, deg1 = _sc_deg(dst)

    lp = p["layers"][0]
    uh, ah, bvlo, bvhi = _node_first(
        x, wt(p["emb_h_w"]), bb(p["emb_h_b"]),
        wt(lp["U_w"]), bb(lp["U_b"]), wt(lp["V_w"]), bb(lp["V_b"]),
        wt(lp["A_w"]), bb(lp["A_b"]), wt(lp["B_w"]), bb(lp["B_b"]))
    ew_t = p["emb_e_w"].reshape(1, D)
    ce = [None, None]
    for half in range(2):
        ce[half] = _edge_first(
            edge_w[half * EH:(half + 1) * EH], ew_t, bb(p["emb_e_b"]),
            wt(lp["C_w"]), bb(lp["C_b"]))
    sasb0 = _sc_edge(0, ce[0][0], ce[0][1], ah, bvlo, bvhi, src, dst)
    sasb1 = _sc_edge(1, ce[1][0], ce[1][1], ah, bvlo, bvhi, src, dst)
    e_lo = [sasb0[2], sasb1[2]]
    e_hi = [sasb0[3], sasb1[3]]
    sasblo0, sasbhi0 = sasb0[0], sasb0[1]
    sasblo1, sasbhi1 = sasb1[0], sasb1[1]

    n_layers = len(p["layers"])
    for li in range(1, n_layers):
        lp = p["layers"][li]
        uh, ah, bvlo, bvhi = _node_rest(
            uh, sasblo0, sasblo1, sasbhi0, sasbhi1, deg0, deg1,
            wt(lp["U_w"]), bb(lp["U_b"]), wt(lp["V_w"]), bb(lp["V_b"]),
            wt(lp["A_w"]), bb(lp["A_b"]), wt(lp["B_w"]), bb(lp["B_b"]))
        for half in range(2):
            ce[half] = _edge_rest(e_lo[half], e_hi[half],
                                  wt(lp["C_w"]), bb(lp["C_b"]))
        if li < n_layers - 1:
            sasb0 = _sc_edge(0, ce[0][0], ce[0][1], ah, bvlo, bvhi, src, dst)
            sasb1 = _sc_edge(1, ce[1][0], ce[1][1], ah, bvlo, bvhi, src, dst)
            e_lo = [sasb0[2], sasb1[2]]
            e_hi = [sasb0[3], sasb1[3]]
        else:
            sasb0 = _sc_edge_last(0, ce[0][0], ce[0][1], ah, bvlo, bvhi,
                                  src, dst)
            sasb1 = _sc_edge_last(1, ce[1][0], ce[1][1], ah, bvlo, bvhi,
                                  src, dst)
        sasblo0, sasbhi0 = sasb0[0], sasb0[1]
        sasblo1, sasbhi1 = sasb1[0], sasb1[1]

    return _fin(uh, sasblo0, sasblo1, sasbhi0, sasbhi1, deg0, deg1)
